# baseline (speedup 1.0000x reference)
import functools

import jax
import jax.numpy as jnp
from jax import lax
from jax.experimental import pallas as pl
from jax.experimental.pallas import tpu as pltpu
from jax.experimental.pallas import tpu_sc as plsc

GX, GY, GZ = 1408, 1600, 40
TOT = GX * GY * GZ
MAXV, MAXP = 16000, 5
N, NPAD = 200000, 200704
NW = 16
PPW = NPAD // NW
NROW = PPW // 128
P1S = 1409024
P1W = P1S // NW
NB_REAL = 1408000
NSLOTS = 16384
T2S = 1081344
T2W = T2S // NW
T2REAL = NSLOTS * 64
CTW = 16384
SLAB = CTW // NW
VWORDS, VDUMPW = 327680, 320000
CWORDS, CDUMPR = 49152, 16128
HALF = PPW // 2
HROW = HALF // 128


def _mesh():
    return plsc.VectorSubcoreMesh(
        core_axis_name="c", subcore_axis_name="s", num_cores=1)


def _iota16():
    return lax.iota(jnp.int32, 16)


def _zero_vec(ref, nwords):
    z = jnp.zeros((16,), ref.dtype)

    def body(i, _):
        ref[pl.ds(pl.multiple_of(i * 16, 16), 16)] = z
        return 0

    lax.fori_loop(0, nwords // 16, body, 0)


def _fire_drain(n, mk, chunk=8):
    for c0 in range(0, n, chunk):
        descs = []
        for j in range(c0, min(n, c0 + chunk)):
            d = mk(j)
            d.start()
            descs.append(d)
        for d in descs:
            d.wait()


def _ids_body(x_ref, y_ref, z_ref, id_ref):
    x = x_ref[...]
    y = y_ref[...]
    z = z_ref[...]
    cx = jnp.floor((x - 0.0) / jnp.float32(0.05)).astype(jnp.int32)
    cy = jnp.floor((y - jnp.float32(-40.0)) / jnp.float32(0.05)).astype(jnp.int32)
    cz = jnp.floor((z - jnp.float32(-3.0)) / jnp.float32(0.1)).astype(jnp.int32)
    ok = (cx >= 0) & (cx < GX) & (cy >= 0) & (cy < GY) & (cz >= 0) & (cz < GZ)
    id_ref[...] = jnp.where(ok, (cz * GY + cy) * GX + cx, TOT)


def _compute_ids(points):
    pad = NPAD - N
    x = jnp.pad(points[:, 0], (0, pad), constant_values=-1.0).reshape(1568, 128)
    y = jnp.pad(points[:, 1], (0, pad)).reshape(1568, 128)
    z = jnp.pad(points[:, 2], (0, pad)).reshape(1568, 128)
    ids = pl.pallas_call(
        _ids_body,
        out_shape=jax.ShapeDtypeStruct((1568, 128), jnp.int32),
    )(x, y, z)
    return ids.reshape(NW, PPW)


def _k2(ids_hbm, p1_hbm, zb, idsv, bkt, ones, sem):
    w = lax.axis_index("s")
    _zero_vec(zb, 2048)
    one = jnp.ones((16,), jnp.int32)
    for k in range(8):
        ones[pl.ds(k * 16, 16)] = one

    base = w * P1W

    def clr(c, _):
        off = pl.multiple_of(base + c * 2048, 8)
        pltpu.sync_copy(zb, p1_hbm.at[pl.ds(off, 2048)])
        return 0

    lax.fori_loop(0, P1W // 2048, clr, 0)
    plsc.subcore_barrier()
    pltpu.sync_copy(ids_hbm.at[w], idsv)

    def outer(j, _):
        def inner(k, _):
            o = pl.multiple_of(j * 128 + k * 16, 16)
            v = idsv[pl.ds(o, 16)]
            bkt[j, pl.ds(pl.multiple_of(k * 16, 16), 16)] = v >> 6
            return 0

        lax.fori_loop(0, 8, inner, 0)
        return 0

    lax.fori_loop(0, NROW, outer, 0)

    def mk(j):
        return pltpu.make_async_copy(ones, p1_hbm.at[bkt.at[j]], sem)

    _fire_drain(NROW, mk)


def _run_k2(ids2d):
    k = pl.kernel(
        _k2,
        out_type=jax.ShapeDtypeStruct((P1S,), jnp.int32),
        mesh=_mesh(),
        compiler_params=pltpu.CompilerParams(needs_layout_passes=False),
        scratch_types=[
            pltpu.VMEM((2048,), jnp.int32),
            pltpu.VMEM((PPW,), jnp.int32),
            pltpu.VMEM((NROW, 128), jnp.int32),
            pltpu.VMEM((128,), jnp.int32),
            pltpu.SemaphoreType.DMA,
        ],
    )
    return k(ids2d)


def _make_count(total_words, per_w, real_limit):
    nchunk = per_w // 2048

    def body(tab_hbm, cnt_hbm, buf, stg):
        w = lax.axis_index("s")
        base = w * per_w
        it = _iota16()

        def outer(c, acc):
            off = pl.multiple_of(base + c * 2048, 8)
            pltpu.sync_copy(tab_hbm.at[pl.ds(off, 2048)], buf)
            gbase = base + c * 2048

            def inner(i, acc):
                v = buf[pl.ds(pl.multiple_of(i * 16, 16), 16)]
                g = gbase + i * 16 + it
                nz = (v != 0) & (g < real_limit)
                return acc + jnp.where(nz, 1, 0)

            return lax.fori_loop(0, 128, inner, acc)

        acc = lax.fori_loop(0, nchunk, outer, jnp.zeros((16,), jnp.int32))
        stg[...] = jnp.broadcast_to(jnp.sum(acc), (16,))
        pltpu.sync_copy(stg, cnt_hbm.at[w])

    def run(tab):
        k = pl.kernel(
            body,
            out_type=jax.ShapeDtypeStruct((16, 16), jnp.int32),
            mesh=_mesh(),
        compiler_params=pltpu.CompilerParams(needs_layout_passes=False),
            scratch_types=[
                pltpu.VMEM((2048,), jnp.int32),
                pltpu.VMEM((16,), jnp.int32),
            ],
        )
        return k(tab)

    return run


_run_k3 = _make_count(P1S, P1W, NB_REAL)
_run_k6 = _make_count(T2S, T2W, T2REAL)


def _worker_prefix(cnt_hbm, cntv, w):
    it = _iota16()
    pltpu.sync_copy(cnt_hbm, cntv)
    totals = plsc.load_gather(cntv, [it, jnp.zeros((16,), jnp.int32)])
    return jnp.sum(jnp.where(it < w, totals, 0))


def _make_compact(total_words, per_w, real_limit, keep, plus_one):
    nchunk = per_w // 2048

    def body(tab_hbm, cnt_hbm, out_hbm, buf, obuf, cntv):
        w = lax.axis_index("s")
        base = w * per_w
        it = _iota16()
        pfx = _worker_prefix(cnt_hbm, cntv, w)

        def outer(c, run):
            off = pl.multiple_of(base + c * 2048, 8)
            pltpu.sync_copy(tab_hbm.at[pl.ds(off, 2048)], buf)
            gbase = base + c * 2048

            def inner(i, run):
                o = pl.ds(pl.multiple_of(i * 16, 16), 16)
                v = buf[o]
                g = gbase + i * 16 + it
                nz = (v != 0) & (g < real_limit)
                nzi = nz.astype(jnp.int32)
                cs = plsc.cumsum(nzi)
                r = run + cs - 1
                kept = nz & (r < keep)
                if plus_one:
                    obuf[o] = jnp.where(kept, r + 1, 0)
                else:
                    obuf[o] = jnp.where(kept, r, -1)
                return run + jnp.sum(nzi)

            run = lax.fori_loop(0, 128, inner, run)
            pltpu.sync_copy(obuf, out_hbm.at[pl.ds(off, 2048)])
            return run

        lax.fori_loop(0, nchunk, outer, pfx)

    def run(tab, cnt):
        k = pl.kernel(
            body,
            out_type=jax.ShapeDtypeStruct((total_words,), jnp.int32),
            mesh=_mesh(),
        compiler_params=pltpu.CompilerParams(needs_layout_passes=False),
            scratch_types=[
                pltpu.VMEM((2048,), jnp.int32),
                pltpu.VMEM((2048,), jnp.int32),
                pltpu.VMEM((16, 16), jnp.int32),
            ],
        )
        return k(tab, cnt)

    return run


_run_k4 = _make_compact(P1S, P1W, NB_REAL, NSLOTS, False)
_run_k7 = _make_compact(T2S, T2W, T2REAL, MAXV, True)


def _k5(ids_hbm, map_hbm, t2_hbm, fidx_hbm, zb, idsv, bkt, mapg, fidxv, valv, sem):
    w = lax.axis_index("s")
    _zero_vec(zb, 2048)
    base2 = w * T2W

    def clr(c, _):
        off = pl.multiple_of(base2 + c * 2048, 8)
        pltpu.sync_copy(zb, t2_hbm.at[pl.ds(off, 2048)])
        return 0

    lax.fori_loop(0, T2W // 2048, clr, 0)
    plsc.subcore_barrier()
    pltpu.sync_copy(ids_hbm.at[w], idsv)

    def outer(j, _):
        def inner(k, _):
            o = pl.multiple_of(j * 128 + k * 16, 16)
            v = idsv[pl.ds(o, 16)]
            bkt[j, pl.ds(pl.multiple_of(k * 16, 16), 16)] = v >> 6
            return 0

        lax.fori_loop(0, 8, inner, 0)
        return 0

    lax.fori_loop(0, NROW, outer, 0)
    _fire_drain(NROW, lambda j: pltpu.make_async_copy(
        map_hbm.at[bkt.at[j]], mapg.at[j], sem))

    def outer2(j, _):
        def inner(k, _):
            o = pl.multiple_of(j * 128 + k * 16, 16)
            ko = pl.ds(pl.multiple_of(k * 16, 16), 16)
            v = idsv[pl.ds(o, 16)]
            s = mapg[j, ko]
            fidxv[j, ko] = jnp.where(s >= 0, s * 64 + (v & 63), T2REAL)
            valv[j, ko] = v + 1
            return 0

        lax.fori_loop(0, 8, inner, 0)
        return 0

    lax.fori_loop(0, NROW, outer2, 0)
    _fire_drain(NROW, lambda j: pltpu.make_async_copy(
        valv.at[j], t2_hbm.at[fidxv.at[j]], sem))
    pltpu.sync_copy(fidxv, fidx_hbm.at[w])


def _run_k5(ids2d, mp):
    k = pl.kernel(
        _k5,
        out_type=(jax.ShapeDtypeStruct((T2S,), jnp.int32),
                  jax.ShapeDtypeStruct((NW, NROW, 128), jnp.int32)),
        mesh=_mesh(),
        compiler_params=pltpu.CompilerParams(needs_layout_passes=False),
        scratch_types=[
            pltpu.VMEM((2048,), jnp.int32),
            pltpu.VMEM((PPW,), jnp.int32),
            pltpu.VMEM((NROW, 128), jnp.int32),
            pltpu.VMEM((NROW, 128), jnp.int32),
            pltpu.VMEM((NROW, 128), jnp.int32),
            pltpu.VMEM((NROW, 128), jnp.int32),
            pltpu.SemaphoreType.DMA,
        ],
    )
    return k(ids2d, mp)


def _k8(fidx_hbm, rows_hbm, packed_hbm, counts_hbm, fidxv, rowsg, cnt_tbl,
        tmp, pkv, sem):
    w = lax.axis_index("s")
    it = _iota16()
    _zero_vec(cnt_tbl, CTW)
    pltpu.sync_copy(fidx_hbm.at[w], fidxv)
    _fire_drain(NROW, lambda j: pltpu.make_async_copy(
        rows_hbm.at[fidxv.at[j]], rowsg.at[j], sem))

    def outer(j, _):
        def inner(k, _):
            ko = pl.ds(pl.multiple_of(k * 16, 16), 16)
            r1 = rowsg[j, ko]
            rt = jnp.where(r1 > 0, r1 - 1, MAXV)
            comp = rt * 16 + it
            sk, _sv = plsc.sort_key_val(comp, comp)
            srow = sk >> 4
            slane = sk & 15
            tmp[...] = srow
            prev = plsc.load_gather(tmp, [jnp.maximum(it - 1, 0)])
            nxt = plsc.load_gather(tmp, [jnp.minimum(it + 1, 15)])
            new_run = (it == 0) | (srow != prev)
            is_last = (it == 15) | (srow != nxt)
            segst = plsc.cummax(jnp.where(new_run, it, 0))
            occ = it - segst
            old = plsc.load_gather(cnt_tbl, [srow])
            lr = old + occ
            plsc.store_scatter(cnt_tbl, [srow], lr + 1, mask=is_last)
            pk = (srow << 14) | lr
            pos = j * 128 + k * 16 + slane
            plsc.store_scatter(pkv, [pos], pk)
            return 0

        lax.fori_loop(0, 8, inner, 0)
        return 0

    lax.fori_loop(0, NROW, outer, 0)
    pltpu.sync_copy(pkv, packed_hbm.at[pl.ds(pl.multiple_of(w * PPW, 8), PPW)])
    pltpu.sync_copy(cnt_tbl, counts_hbm.at[w])


def _run_k8(fidx, t2rows):
    k = pl.kernel(
        _k8,
        out_type=(jax.ShapeDtypeStruct((NW * PPW,), jnp.int32),
                  jax.ShapeDtypeStruct((NW, CTW), jnp.int32)),
        mesh=_mesh(),
        compiler_params=pltpu.CompilerParams(needs_layout_passes=False),
        scratch_types=[
            pltpu.VMEM((NROW, 128), jnp.int32),
            pltpu.VMEM((NROW, 128), jnp.int32),
            pltpu.VMEM((CTW,), jnp.int32),
            pltpu.VMEM((16,), jnp.int32),
            pltpu.VMEM((PPW,), jnp.int32),
            pltpu.SemaphoreType.DMA,
        ],
    )
    return k(fidx, t2rows)


def _k9(counts_hbm, prefix_hbm, nump_hbm, accv, cb):
    w = lax.axis_index("s")
    sb = w * SLAB
    _zero_vec(accv, SLAB)

    for v in range(NW):
        off = pl.multiple_of(v * CTW + sb, 8)
        pltpu.sync_copy(accv, prefix_hbm.at[pl.ds(off, SLAB)])
        pltpu.sync_copy(counts_hbm.at[v].at[pl.ds(pl.multiple_of(sb, 8), SLAB)], cb)

        def add(i, _):
            o = pl.ds(pl.multiple_of(i * 16, 16), 16)
            accv[o] = accv[o] + cb[o]
            return 0

        lax.fori_loop(0, SLAB // 16, add, 0)

    def fin(i, _):
        o = pl.ds(pl.multiple_of(i * 16, 16), 16)
        cb[o] = jnp.minimum(accv[o], MAXP)
        return 0

    lax.fori_loop(0, SLAB // 16, fin, 0)
    pltpu.sync_copy(cb, nump_hbm.at[pl.ds(pl.multiple_of(sb, 8), SLAB)])


def _run_k9(counts):
    k = pl.kernel(
        _k9,
        out_type=(jax.ShapeDtypeStruct((NW * CTW,), jnp.int32),
                  jax.ShapeDtypeStruct((CTW,), jnp.int32)),
        mesh=_mesh(),
        compiler_params=pltpu.CompilerParams(needs_layout_passes=False),
        scratch_types=[
            pltpu.VMEM((SLAB,), jnp.int32),
            pltpu.VMEM((SLAB,), jnp.int32),
        ],
    )
    return k(counts)


def _k10(pts_hbm, ids_hbm, packed_hbm, prefix_hbm, vox_hbm, coors_hbm,
         zbf, zbi, pc0, pc1, pc2, pc3, idsv, pkv, pgi, pg,
         vi0, vi1, vi2, vi3, ci0, ci1, ci2, cv0, cv1, cv2, sem):
    w = lax.axis_index("s")
    it = _iota16()
    _zero_vec(zbf, 2048)
    _zero_vec(zbi, 1024)
    for c in range(10):
        pltpu.sync_copy(zbf, vox_hbm.at[pl.ds(
            pl.multiple_of(w * 20480 + c * 2048, 8), 2048)])
    for c in range(3):
        pltpu.sync_copy(zbi, coors_hbm.at[pl.ds(
            pl.multiple_of(w * 3072 + c * 1024, 8), 1024)])
    plsc.subcore_barrier()

    pcs = (pc0, pc1, pc2, pc3)
    vis = (vi0, vi1, vi2, vi3)
    cis = (ci0, ci1, ci2)
    cvs = (cv0, cv1, cv2)
    for h in range(2):
        base = w * PPW + h * HALF
        off = pl.ds(pl.multiple_of(base, 8), HALF)
        for c in range(4):
            pltpu.sync_copy(pts_hbm.at[c].at[off], pcs[c])
        pltpu.sync_copy(
            ids_hbm.at[w].at[pl.ds(pl.multiple_of(h * HALF, 8), HALF)], idsv)
        pltpu.sync_copy(packed_hbm.at[off], pkv)

        def bld1(j, _):
            def inner(k, _):
                o = pl.ds(pl.multiple_of(j * 128 + k * 16, 16), 16)
                ko = pl.ds(pl.multiple_of(k * 16, 16), 16)
                pk = pkv[o]
                pgi[j, ko] = w * CTW + (pk >> 14)
                return 0

            lax.fori_loop(0, 8, inner, 0)
            return 0

        lax.fori_loop(0, HROW, bld1, 0)
        _fire_drain(HROW, lambda j: pltpu.make_async_copy(
            prefix_hbm.at[pgi.at[j]], pg.at[j], sem))

        def bld2(j, _):
            def inner(k, _):
                o = pl.ds(pl.multiple_of(j * 128 + k * 16, 16), 16)
                ko = pl.ds(pl.multiple_of(k * 16, 16), 16)
                pk = pkv[o]
                rt = pk >> 14
                g = pg[j, ko] + (pk & 16383)
                okr = rt < MAXV
                keepv = okr & (g < MAXP)
                vf = jnp.where(keepv, (rt * 5 + g) * 4, VDUMPW)
                vi0[j, ko] = vf
                vi1[j, ko] = vf + 1
                vi2[j, ko] = vf + 2
                vi3[j, ko] = vf + 3
                cf = jnp.where(okr, rt * 3, CDUMPR * 3)
                ci0[j, ko] = cf
                ci1[j, ko] = cf + 1
                ci2[j, ko] = cf + 2
                idv = idsv[o]
                cxv = idv % GX
                rr = idv // GX
                cv0[o] = rr // GY
                cv1[o] = rr % GY
                cv2[o] = cxv
                return 0

            lax.fori_loop(0, 8, inner, 0)
            return 0

        lax.fori_loop(0, HROW, bld2, 0)
        for c in range(4):
            vic = vis[c]
            pcc = pcs[c]
            _fire_drain(HROW, lambda j: pltpu.make_async_copy(
                pcc.at[pl.ds(j * 128, 128)], vox_hbm.at[vic.at[j]], sem))
        for c in range(3):
            cic = cis[c]
            cvc = cvs[c]
            _fire_drain(HROW, lambda j: pltpu.make_async_copy(
                cvc.at[pl.ds(j * 128, 128)], coors_hbm.at[cic.at[j]], sem))


def _run_k10(pts_t, ids2d, packed, prefix):
    k = pl.kernel(
        _k10,
        out_type=(jax.ShapeDtypeStruct((VWORDS,), jnp.float32),
                  jax.ShapeDtypeStruct((CWORDS,), jnp.int32)),
        mesh=_mesh(),
        compiler_params=pltpu.CompilerParams(needs_layout_passes=False),
        scratch_types=(
            [pltpu.VMEM((2048,), jnp.float32),
             pltpu.VMEM((1024,), jnp.int32)]
            + [pltpu.VMEM((HALF,), jnp.float32) for _ in range(4)]
            + [pltpu.VMEM((HALF,), jnp.int32),
               pltpu.VMEM((HALF,), jnp.int32)]
            + [pltpu.VMEM((HROW, 128), jnp.int32) for _ in range(2)]
            + [pltpu.VMEM((HROW, 128), jnp.int32) for _ in range(7)]
            + [pltpu.VMEM((HALF,), jnp.int32) for _ in range(3)]
            + [pltpu.SemaphoreType.DMA]
        ),
    )
    return k(pts_t, ids2d, packed, prefix)


def kernel(input):
    points = input
    ids2d = _compute_ids(points)
    p1 = _run_k2(ids2d)
    cnt = _run_k3(p1)
    mp = _run_k4(p1, cnt)
    t2, fidx = _run_k5(ids2d, mp)
    fcnt = _run_k6(t2)
    t2rows = _run_k7(t2, fcnt)
    packed, counts = _run_k8(fidx, t2rows)
    prefix, nump_buf = _run_k9(counts)
    pts_t = jnp.pad(points, ((0, NPAD - N), (0, 0))).T.reshape(4, NPAD)
    vox_flat, coors_flat = _run_k10(pts_t, ids2d, packed, prefix)
    voxels = vox_flat[:MAXV * MAXP * 4].reshape(MAXV, MAXP, 4)
    coors = coors_flat.reshape(CWORDS // 3, 3)[:MAXV]
    return voxels, coors, nump_buf[:MAXV]


# speedup vs baseline: 37.4812x; 37.4812x over previous
import functools

import jax
import jax.numpy as jnp
from jax import lax
from jax.experimental import pallas as pl
from jax.experimental.pallas import tpu as pltpu
from jax.experimental.pallas import tpu_sc as plsc

GX, GY, GZ = 1408, 1600, 40
TOT = GX * GY * GZ
MAXV, MAXP = 16000, 5
N, NPAD = 200000, 200704
NW = 16
PPW = NPAD // NW
NROW = PPW // 128
P1S = 1409024
P1W = P1S // NW
NB_REAL = 1408000
NSLOTS = 16384
T2S = 1081344
T2W = T2S // NW
T2REAL = NSLOTS * 64
CTW = 16384
SLAB = CTW // NW
VWORDS, VDUMPW = 360448, 320000
CWORDS, CDUMPW = 81920, 48384
HALF = PPW // 2
HROW = HALF // 128


def _mesh():
    return plsc.VectorSubcoreMesh(
        core_axis_name="c", subcore_axis_name="s", num_cores=1)


def _iota16():
    return lax.iota(jnp.int32, 16)


def _zero_vec(ref, nwords):
    z = jnp.zeros((16,), ref.dtype)

    def body(i, _):
        ref[pl.ds(pl.multiple_of(i * 16, 16), 16)] = z
        return 0

    lax.fori_loop(0, nwords // 16, body, 0)


def _fire_drain(n, mk, chunk=8):
    for c0 in range(0, n, chunk):
        descs = []
        for j in range(c0, min(n, c0 + chunk)):
            d = mk(j)
            d.start()
            descs.append(d)
        for d in descs:
            d.wait()


def _ids_body(x_ref, y_ref, z_ref, id_ref):
    x = x_ref[...]
    y = y_ref[...]
    z = z_ref[...]
    cx = jnp.floor((x - 0.0) / jnp.float32(0.05)).astype(jnp.int32)
    cy = jnp.floor((y - jnp.float32(-40.0)) / jnp.float32(0.05)).astype(jnp.int32)
    cz = jnp.floor((z - jnp.float32(-3.0)) / jnp.float32(0.1)).astype(jnp.int32)
    ok = (cx >= 0) & (cx < GX) & (cy >= 0) & (cy < GY) & (cz >= 0) & (cz < GZ)
    id_ref[...] = jnp.where(ok, (cz * GY + cy) * GX + cx, TOT)


def _compute_ids(points):
    pad = NPAD - N
    x = jnp.pad(points[:, 0], (0, pad), constant_values=-1.0).reshape(1568, 128)
    y = jnp.pad(points[:, 1], (0, pad)).reshape(1568, 128)
    z = jnp.pad(points[:, 2], (0, pad)).reshape(1568, 128)
    ids = pl.pallas_call(
        _ids_body,
        out_shape=jax.ShapeDtypeStruct((1568, 128), jnp.int32),
    )(x, y, z)
    return ids.reshape(NW, PPW)


def _k2(ids_hbm, p1_hbm, zb, idsv, bkt, ones, sem):
    w = lax.axis_index("s")
    _zero_vec(zb, 2048)
    one = jnp.ones((16,), jnp.int32)
    for k in range(8):
        ones[pl.ds(k * 16, 16)] = one

    base = w * P1W

    def clr(c, _):
        off = pl.multiple_of(base + c * 2048, 8)
        pltpu.sync_copy(zb, p1_hbm.at[pl.ds(off, 2048)])
        return 0

    lax.fori_loop(0, P1W // 2048, clr, 0)
    plsc.subcore_barrier()
    pltpu.sync_copy(ids_hbm.at[w], idsv)

    def outer(j, _):
        def inner(k, _):
            o = pl.multiple_of(j * 128 + k * 16, 16)
            v = idsv[pl.ds(o, 16)]
            bkt[j, pl.ds(pl.multiple_of(k * 16, 16), 16)] = v >> 6
            return 0

        lax.fori_loop(0, 8, inner, 0)
        return 0

    lax.fori_loop(0, NROW, outer, 0)

    def mk(j):
        return pltpu.make_async_copy(ones, p1_hbm.at[bkt.at[j]], sem)

    _fire_drain(NROW, mk)


def _run_k2(ids2d):
    k = pl.kernel(
        _k2,
        out_type=jax.ShapeDtypeStruct((P1S,), jnp.int32),
        mesh=_mesh(),
        compiler_params=pltpu.CompilerParams(needs_layout_passes=False),
        scratch_types=[
            pltpu.VMEM((2048,), jnp.int32),
            pltpu.VMEM((PPW,), jnp.int32),
            pltpu.VMEM((NROW, 128), jnp.int32),
            pltpu.VMEM((128,), jnp.int32),
            pltpu.SemaphoreType.DMA,
        ],
    )
    return k(ids2d)


def _make_count(total_words, per_w, real_limit):
    nchunk = per_w // 2048

    def body(tab_hbm, cnt_hbm, buf, stg):
        w = lax.axis_index("s")
        base = w * per_w
        it = _iota16()

        def outer(c, acc):
            off = pl.multiple_of(base + c * 2048, 8)
            pltpu.sync_copy(tab_hbm.at[pl.ds(off, 2048)], buf)
            gbase = base + c * 2048

            def inner(i, acc):
                v = buf[pl.ds(pl.multiple_of(i * 16, 16), 16)]
                g = gbase + i * 16 + it
                nz = (v != 0) & (g < real_limit)
                return acc + jnp.where(nz, 1, 0)

            return lax.fori_loop(0, 128, inner, acc)

        acc = lax.fori_loop(0, nchunk, outer, jnp.zeros((16,), jnp.int32))
        stg[...] = jnp.broadcast_to(jnp.sum(acc), (16,))
        pltpu.sync_copy(stg, cnt_hbm.at[w])

    def run(tab):
        k = pl.kernel(
            body,
            out_type=jax.ShapeDtypeStruct((16, 16), jnp.int32),
            mesh=_mesh(),
        compiler_params=pltpu.CompilerParams(needs_layout_passes=False),
            scratch_types=[
                pltpu.VMEM((2048,), jnp.int32),
                pltpu.VMEM((16,), jnp.int32),
            ],
        )
        return k(tab)

    return run


_run_k3 = _make_count(P1S, P1W, NB_REAL)
_run_k6 = _make_count(T2S, T2W, T2REAL)


def _worker_prefix(cnt_hbm, cntv, w):
    it = _iota16()
    pltpu.sync_copy(cnt_hbm, cntv)
    totals = plsc.load_gather(cntv, [it, jnp.zeros((16,), jnp.int32)])
    return jnp.sum(jnp.where(it < w, totals, 0))


def _make_compact(total_words, per_w, real_limit, keep, plus_one):
    nchunk = per_w // 2048

    def body(tab_hbm, cnt_hbm, out_hbm, buf, obuf, cntv):
        w = lax.axis_index("s")
        base = w * per_w
        it = _iota16()
        pfx = _worker_prefix(cnt_hbm, cntv, w)

        def outer(c, run):
            off = pl.multiple_of(base + c * 2048, 8)
            pltpu.sync_copy(tab_hbm.at[pl.ds(off, 2048)], buf)
            gbase = base + c * 2048

            def inner(i, run):
                o = pl.ds(pl.multiple_of(i * 16, 16), 16)
                v = buf[o]
                g = gbase + i * 16 + it
                nz = (v != 0) & (g < real_limit)
                nzi = nz.astype(jnp.int32)
                cs = plsc.cumsum(nzi)
                r = run + cs - 1
                kept = nz & (r < keep)
                if plus_one:
                    obuf[o] = jnp.where(kept, r + 1, 0)
                else:
                    obuf[o] = jnp.where(kept, r, -1)
                return run + jnp.sum(nzi)

            run = lax.fori_loop(0, 128, inner, run)
            pltpu.sync_copy(obuf, out_hbm.at[pl.ds(off, 2048)])
            return run

        lax.fori_loop(0, nchunk, outer, pfx)

    def run(tab, cnt):
        k = pl.kernel(
            body,
            out_type=jax.ShapeDtypeStruct((total_words,), jnp.int32),
            mesh=_mesh(),
        compiler_params=pltpu.CompilerParams(needs_layout_passes=False),
            scratch_types=[
                pltpu.VMEM((2048,), jnp.int32),
                pltpu.VMEM((2048,), jnp.int32),
                pltpu.VMEM((16, 16), jnp.int32),
            ],
        )
        return k(tab, cnt)

    return run


_run_k4 = _make_compact(P1S, P1W, NB_REAL, NSLOTS, False)
_run_k7 = _make_compact(T2S, T2W, T2REAL, MAXV, True)


def _k5(ids_hbm, map_hbm, t2_hbm, fidx_hbm, zb, idsv, bkt, mapg, fidxv, valv, sem):
    w = lax.axis_index("s")
    _zero_vec(zb, 2048)
    base2 = w * T2W

    def clr(c, _):
        off = pl.multiple_of(base2 + c * 2048, 8)
        pltpu.sync_copy(zb, t2_hbm.at[pl.ds(off, 2048)])
        return 0

    lax.fori_loop(0, T2W // 2048, clr, 0)
    plsc.subcore_barrier()
    pltpu.sync_copy(ids_hbm.at[w], idsv)

    def outer(j, _):
        def inner(k, _):
            o = pl.multiple_of(j * 128 + k * 16, 16)
            v = idsv[pl.ds(o, 16)]
            bkt[j, pl.ds(pl.multiple_of(k * 16, 16), 16)] = v >> 6
            return 0

        lax.fori_loop(0, 8, inner, 0)
        return 0

    lax.fori_loop(0, NROW, outer, 0)
    _fire_drain(NROW, lambda j: pltpu.make_async_copy(
        map_hbm.at[bkt.at[j]], mapg.at[j], sem))

    def outer2(j, _):
        def inner(k, _):
            o = pl.multiple_of(j * 128 + k * 16, 16)
            ko = pl.ds(pl.multiple_of(k * 16, 16), 16)
            v = idsv[pl.ds(o, 16)]
            s = mapg[j, ko]
            pos = j * 128 + k * 16 + _iota16()
            fidxv[j, ko] = jnp.where(s >= 0, s * 64 + (v & 63), T2REAL + pos)
            valv[j, ko] = v + 1
            return 0

        lax.fori_loop(0, 8, inner, 0)
        return 0

    lax.fori_loop(0, NROW, outer2, 0)
    _fire_drain(NROW, lambda j: pltpu.make_async_copy(
        valv.at[j], t2_hbm.at[fidxv.at[j]], sem))
    pltpu.sync_copy(fidxv, fidx_hbm.at[w])


def _run_k5(ids2d, mp):
    k = pl.kernel(
        _k5,
        out_type=(jax.ShapeDtypeStruct((T2S,), jnp.int32),
                  jax.ShapeDtypeStruct((NW, NROW, 128), jnp.int32)),
        mesh=_mesh(),
        compiler_params=pltpu.CompilerParams(needs_layout_passes=False),
        scratch_types=[
            pltpu.VMEM((2048,), jnp.int32),
            pltpu.VMEM((PPW,), jnp.int32),
            pltpu.VMEM((NROW, 128), jnp.int32),
            pltpu.VMEM((NROW, 128), jnp.int32),
            pltpu.VMEM((NROW, 128), jnp.int32),
            pltpu.VMEM((NROW, 128), jnp.int32),
            pltpu.SemaphoreType.DMA,
        ],
    )
    return k(ids2d, mp)


def _k8(fidx_hbm, rows_hbm, packed_hbm, counts_hbm, fidxv, rowsg, cnt_tbl,
        tmp, pkv, sem):
    w = lax.axis_index("s")
    it = _iota16()
    _zero_vec(cnt_tbl, CTW)
    pltpu.sync_copy(fidx_hbm.at[w], fidxv)
    _fire_drain(NROW, lambda j: pltpu.make_async_copy(
        rows_hbm.at[fidxv.at[j]], rowsg.at[j], sem))

    def outer(j, _):
        def inner(k, _):
            ko = pl.ds(pl.multiple_of(k * 16, 16), 16)
            r1 = rowsg[j, ko]
            rt = jnp.where(r1 > 0, r1 - 1, MAXV)
            comp = rt * 16 + it
            sk, _sv = plsc.sort_key_val(comp, comp)
            srow = sk >> 4
            slane = sk & 15
            tmp[...] = srow
            prev = plsc.load_gather(tmp, [jnp.maximum(it - 1, 0)])
            nxt = plsc.load_gather(tmp, [jnp.minimum(it + 1, 15)])
            new_run = (it == 0) | (srow != prev)
            is_last = (it == 15) | (srow != nxt)
            segst = plsc.cummax(jnp.where(new_run, it, 0))
            occ = it - segst
            old = plsc.load_gather(cnt_tbl, [srow])
            lr = old + occ
            plsc.store_scatter(cnt_tbl, [srow], lr + 1, mask=is_last)
            pk = (srow << 14) | lr
            pos = j * 128 + k * 16 + slane
            plsc.store_scatter(pkv, [pos], pk)
            return 0

        lax.fori_loop(0, 8, inner, 0)
        return 0

    lax.fori_loop(0, NROW, outer, 0)
    pltpu.sync_copy(pkv, packed_hbm.at[pl.ds(pl.multiple_of(w * PPW, 8), PPW)])
    pltpu.sync_copy(cnt_tbl, counts_hbm.at[w])


def _run_k8(fidx, t2rows):
    k = pl.kernel(
        _k8,
        out_type=(jax.ShapeDtypeStruct((NW * PPW,), jnp.int32),
                  jax.ShapeDtypeStruct((NW, CTW), jnp.int32)),
        mesh=_mesh(),
        compiler_params=pltpu.CompilerParams(needs_layout_passes=False),
        scratch_types=[
            pltpu.VMEM((NROW, 128), jnp.int32),
            pltpu.VMEM((NROW, 128), jnp.int32),
            pltpu.VMEM((CTW,), jnp.int32),
            pltpu.VMEM((16,), jnp.int32),
            pltpu.VMEM((PPW,), jnp.int32),
            pltpu.SemaphoreType.DMA,
        ],
    )
    return k(fidx, t2rows)


def _k9(counts_hbm, prefix_hbm, nump_hbm, accv, cb):
    w = lax.axis_index("s")
    sb = w * SLAB
    _zero_vec(accv, SLAB)

    for v in range(NW):
        off = pl.multiple_of(v * CTW + sb, 8)
        pltpu.sync_copy(accv, prefix_hbm.at[pl.ds(off, SLAB)])
        pltpu.sync_copy(counts_hbm.at[v].at[pl.ds(pl.multiple_of(sb, 8), SLAB)], cb)

        def add(i, _):
            o = pl.ds(pl.multiple_of(i * 16, 16), 16)
            accv[o] = accv[o] + cb[o]
            return 0

        lax.fori_loop(0, SLAB // 16, add, 0)

    def fin(i, _):
        o = pl.ds(pl.multiple_of(i * 16, 16), 16)
        cb[o] = jnp.minimum(accv[o], MAXP)
        return 0

    lax.fori_loop(0, SLAB // 16, fin, 0)
    pltpu.sync_copy(cb, nump_hbm.at[pl.ds(pl.multiple_of(sb, 8), SLAB)])


def _run_k9(counts):
    k = pl.kernel(
        _k9,
        out_type=(jax.ShapeDtypeStruct((NW * CTW,), jnp.int32),
                  jax.ShapeDtypeStruct((CTW,), jnp.int32)),
        mesh=_mesh(),
        compiler_params=pltpu.CompilerParams(needs_layout_passes=False),
        scratch_types=[
            pltpu.VMEM((SLAB,), jnp.int32),
            pltpu.VMEM((SLAB,), jnp.int32),
        ],
    )
    return k(counts)


def _k10(pts_hbm, ids_hbm, packed_hbm, prefix_hbm, vox_hbm, coors_hbm,
         zbf, zbi, pc0, pc1, pc2, pc3, idsv, pkv, pgi, pg,
         vi0, vi1, vi2, vi3, ci0, ci1, ci2, cv0, cv1, cv2, sem):
    w = lax.axis_index("s")
    it = _iota16()
    _zero_vec(zbf, 2048)
    _zero_vec(zbi, 1024)
    for c in range(11):
        pltpu.sync_copy(zbf, vox_hbm.at[pl.ds(
            pl.multiple_of(w * 22528 + c * 2048, 8), 2048)])
    for c in range(5):
        pltpu.sync_copy(zbi, coors_hbm.at[pl.ds(
            pl.multiple_of(w * 5120 + c * 1024, 8), 1024)])
    plsc.subcore_barrier()

    pcs = (pc0, pc1, pc2, pc3)
    vis = (vi0, vi1, vi2, vi3)
    cis = (ci0, ci1, ci2)
    cvs = (cv0, cv1, cv2)
    for h in range(2):
        base = w * PPW + h * HALF
        off = pl.ds(pl.multiple_of(base, 8), HALF)
        for c in range(4):
            pltpu.sync_copy(pts_hbm.at[c].at[off], pcs[c])
        pltpu.sync_copy(
            ids_hbm.at[w].at[pl.ds(pl.multiple_of(h * HALF, 8), HALF)], idsv)
        pltpu.sync_copy(packed_hbm.at[off], pkv)

        def bld1(j, _):
            def inner(k, _):
                o = pl.ds(pl.multiple_of(j * 128 + k * 16, 16), 16)
                ko = pl.ds(pl.multiple_of(k * 16, 16), 16)
                pk = pkv[o]
                pgi[j, ko] = w * CTW + (pk >> 14)
                return 0

            lax.fori_loop(0, 8, inner, 0)
            return 0

        lax.fori_loop(0, HROW, bld1, 0)
        _fire_drain(HROW, lambda j: pltpu.make_async_copy(
            prefix_hbm.at[pgi.at[j]], pg.at[j], sem))

        def bld2(j, _):
            def inner(k, _):
                o = pl.ds(pl.multiple_of(j * 128 + k * 16, 16), 16)
                ko = pl.ds(pl.multiple_of(k * 16, 16), 16)
                pk = pkv[o]
                rt = pk >> 14
                g = pg[j, ko] + (pk & 16383)
                okr = rt < MAXV
                keepv = okr & (g < MAXP)
                pos = j * 128 + k * 16 + it
                vf = jnp.where(keepv, (rt * 5 + g) * 4, VDUMPW + pos * 4)
                vi0[j, ko] = vf
                vi1[j, ko] = vf + 1
                vi2[j, ko] = vf + 2
                vi3[j, ko] = vf + 3
                cf = jnp.where(okr, rt * 3, CDUMPW + pos * 3)
                ci0[j, ko] = cf
                ci1[j, ko] = cf + 1
                ci2[j, ko] = cf + 2
                idv = idsv[o]
                cxv = idv % GX
                rr = idv // GX
                cv0[o] = rr // GY
                cv1[o] = rr % GY
                cv2[o] = cxv
                return 0

            lax.fori_loop(0, 8, inner, 0)
            return 0

        lax.fori_loop(0, HROW, bld2, 0)
        for c in range(4):
            vic = vis[c]
            pcc = pcs[c]
            _fire_drain(HROW, lambda j: pltpu.make_async_copy(
                pcc.at[pl.ds(j * 128, 128)], vox_hbm.at[vic.at[j]], sem))
        for c in range(3):
            cic = cis[c]
            cvc = cvs[c]
            _fire_drain(HROW, lambda j: pltpu.make_async_copy(
                cvc.at[pl.ds(j * 128, 128)], coors_hbm.at[cic.at[j]], sem))


def _run_k10(pts_t, ids2d, packed, prefix):
    k = pl.kernel(
        _k10,
        out_type=(jax.ShapeDtypeStruct((VWORDS,), jnp.float32),
                  jax.ShapeDtypeStruct((CWORDS,), jnp.int32)),
        mesh=_mesh(),
        compiler_params=pltpu.CompilerParams(needs_layout_passes=False),
        scratch_types=(
            [pltpu.VMEM((2048,), jnp.float32),
             pltpu.VMEM((1024,), jnp.int32)]
            + [pltpu.VMEM((HALF,), jnp.float32) for _ in range(4)]
            + [pltpu.VMEM((HALF,), jnp.int32),
               pltpu.VMEM((HALF,), jnp.int32)]
            + [pltpu.VMEM((HROW, 128), jnp.int32) for _ in range(2)]
            + [pltpu.VMEM((HROW, 128), jnp.int32) for _ in range(7)]
            + [pltpu.VMEM((HALF,), jnp.int32) for _ in range(3)]
            + [pltpu.SemaphoreType.DMA]
        ),
    )
    return k(pts_t, ids2d, packed, prefix)


def kernel(input):
    points = input
    ids2d = _compute_ids(points)
    p1 = _run_k2(ids2d)
    cnt = _run_k3(p1)
    mp = _run_k4(p1, cnt)
    t2, fidx = _run_k5(ids2d, mp)
    fcnt = _run_k6(t2)
    t2rows = _run_k7(t2, fcnt)
    packed, counts = _run_k8(fidx, t2rows)
    prefix, nump_buf = _run_k9(counts)
    pts_t = jnp.pad(points, ((0, NPAD - N), (0, 0))).T.reshape(4, NPAD)
    vox_flat, coors_flat = _run_k10(pts_t, ids2d, packed, prefix)
    voxels = vox_flat[:MAXV * MAXP * 4].reshape(MAXV, MAXP, 4)
    coors = coors_flat[:MAXV * 3].reshape(MAXV, 3)
    return voxels, coors, nump_buf[:MAXV]


# speedup vs baseline: 51.4229x; 1.3720x over previous
import functools

import jax
import jax.numpy as jnp
from jax import lax
from jax.experimental import pallas as pl
from jax.experimental.pallas import tpu as pltpu
from jax.experimental.pallas import tpu_sc as plsc

GX, GY, GZ = 1408, 1600, 40
TOT = GX * GY * GZ
MAXV, MAXP = 16000, 5
N, NPAD = 200000, 200704
NW = 16
PPW = NPAD // NW
NROW = PPW // 128
P1S = 1409024
P1W = P1S // NW
NB_REAL = 1408000
NSLOTS = 16384
T2S = 1277952
T2W = T2S // NW
T2REAL = NSLOTS * 64
CTW = 16384
SLAB = CTW // NW
VWORDS, VDUMPW = 1146880, 320000
CWORDS, CDUMPW = 655360, 48384
HALF = PPW // 2
HROW = HALF // 128


def _mesh():
    return plsc.VectorSubcoreMesh(
        core_axis_name="c", subcore_axis_name="s", num_cores=1)


def _iota16():
    return lax.iota(jnp.int32, 16)


def _zero_vec(ref, nwords):
    z = jnp.zeros((16,), ref.dtype)

    def body(i, _):
        ref[pl.ds(pl.multiple_of(i * 16, 16), 16)] = z
        return 0

    lax.fori_loop(0, nwords // 16, body, 0)


def _fire_drain(n, mk, chunk=8):
    for c0 in range(0, n, chunk):
        descs = []
        for j in range(c0, min(n, c0 + chunk)):
            d = mk(j)
            d.start()
            descs.append(d)
        for d in descs:
            d.wait()


def _ids_body(x_ref, y_ref, z_ref, id_ref, pc_ref):
    x = x_ref[...]
    y = y_ref[...]
    z = z_ref[...]
    cx = jnp.floor((x - 0.0) / jnp.float32(0.05)).astype(jnp.int32)
    cy = jnp.floor((y - jnp.float32(-40.0)) / jnp.float32(0.05)).astype(jnp.int32)
    cz = jnp.floor((z - jnp.float32(-3.0)) / jnp.float32(0.1)).astype(jnp.int32)
    ok = (cx >= 0) & (cx < GX) & (cy >= 0) & (cy < GY) & (cz >= 0) & (cz < GZ)
    id_ref[...] = jnp.where(ok, (cz * GY + cy) * GX + cx, TOT)
    pc_ref[...] = (cz << 22) | (cy << 11) | cx


def _compute_ids(points):
    pad = NPAD - N
    x = jnp.pad(points[:, 0], (0, pad), constant_values=-1.0).reshape(1568, 128)
    y = jnp.pad(points[:, 1], (0, pad)).reshape(1568, 128)
    z = jnp.pad(points[:, 2], (0, pad)).reshape(1568, 128)
    ids, pc = pl.pallas_call(
        _ids_body,
        out_shape=(jax.ShapeDtypeStruct((1568, 128), jnp.int32),
                   jax.ShapeDtypeStruct((1568, 128), jnp.int32)),
    )(x, y, z)
    return ids.reshape(NW, PPW), pc.reshape(NW, PPW)


def _k2(ids_hbm, p1_hbm, zb, idsv, bkt, ones, sem):
    w = lax.axis_index("s")
    _zero_vec(zb, 2048)
    one = jnp.ones((16,), jnp.int32)
    for k in range(8):
        ones[pl.ds(k * 16, 16)] = one

    base = w * P1W

    def clr(c, _):
        off = pl.multiple_of(base + c * 2048, 8)
        pltpu.sync_copy(zb, p1_hbm.at[pl.ds(off, 2048)])
        return 0

    lax.fori_loop(0, P1W // 2048, clr, 0)
    plsc.subcore_barrier()
    pltpu.sync_copy(ids_hbm.at[w], idsv)

    def outer(j, _):
        def inner(k, _):
            o = pl.multiple_of(j * 128 + k * 16, 16)
            v = idsv[pl.ds(o, 16)]
            bkt[j, pl.ds(pl.multiple_of(k * 16, 16), 16)] = v >> 6
            return 0

        lax.fori_loop(0, 8, inner, 0)
        return 0

    lax.fori_loop(0, NROW, outer, 0)

    def mk(j):
        return pltpu.make_async_copy(ones, p1_hbm.at[bkt.at[j]], sem)

    _fire_drain(NROW, mk)


def _run_k2(ids2d):
    k = pl.kernel(
        _k2,
        out_type=jax.ShapeDtypeStruct((P1S,), jnp.int32),
        mesh=_mesh(),
        compiler_params=pltpu.CompilerParams(needs_layout_passes=False),
        scratch_types=[
            pltpu.VMEM((2048,), jnp.int32),
            pltpu.VMEM((PPW,), jnp.int32),
            pltpu.VMEM((NROW, 128), jnp.int32),
            pltpu.VMEM((128,), jnp.int32),
            pltpu.SemaphoreType.DMA,
        ],
    )
    return k(ids2d)


def _make_count(total_words, per_w, real_limit):
    nchunk = per_w // 2048

    def body(tab_hbm, cnt_hbm, buf, stg):
        w = lax.axis_index("s")
        base = w * per_w
        it = _iota16()

        def outer(c, acc):
            off = pl.multiple_of(base + c * 2048, 8)
            pltpu.sync_copy(tab_hbm.at[pl.ds(off, 2048)], buf)
            gbase = base + c * 2048

            def inner(i, acc):
                v = buf[pl.ds(pl.multiple_of(i * 16, 16), 16)]
                g = gbase + i * 16 + it
                nz = (v != 0) & (g < real_limit)
                return acc + jnp.where(nz, 1, 0)

            return lax.fori_loop(0, 128, inner, acc)

        acc = lax.fori_loop(0, nchunk, outer, jnp.zeros((16,), jnp.int32))
        stg[...] = jnp.broadcast_to(jnp.sum(acc), (16,))
        pltpu.sync_copy(stg, cnt_hbm.at[w])

    def run(tab):
        k = pl.kernel(
            body,
            out_type=jax.ShapeDtypeStruct((16, 16), jnp.int32),
            mesh=_mesh(),
        compiler_params=pltpu.CompilerParams(needs_layout_passes=False),
            scratch_types=[
                pltpu.VMEM((2048,), jnp.int32),
                pltpu.VMEM((16,), jnp.int32),
            ],
        )
        return k(tab)

    return run


_run_k3 = _make_count(P1S, P1W, NB_REAL)
_run_k6 = _make_count(T2S, T2W, T2REAL)


def _worker_prefix(cnt_hbm, cntv, w):
    it = _iota16()
    pltpu.sync_copy(cnt_hbm, cntv)
    totals = plsc.load_gather(cntv, [it, jnp.zeros((16,), jnp.int32)])
    return jnp.sum(jnp.where(it < w, totals, 0))


def _make_compact(total_words, per_w, real_limit, keep, plus_one):
    nchunk = per_w // 2048

    def body(tab_hbm, cnt_hbm, out_hbm, buf, obuf, cntv):
        w = lax.axis_index("s")
        base = w * per_w
        it = _iota16()
        pfx = _worker_prefix(cnt_hbm, cntv, w)

        def outer(c, run):
            off = pl.multiple_of(base + c * 2048, 8)
            pltpu.sync_copy(tab_hbm.at[pl.ds(off, 2048)], buf)
            gbase = base + c * 2048

            def inner(i, run):
                o = pl.ds(pl.multiple_of(i * 16, 16), 16)
                v = buf[o]
                g = gbase + i * 16 + it
                nz = (v != 0) & (g < real_limit)
                nzi = nz.astype(jnp.int32)
                cs = plsc.cumsum(nzi)
                r = run + cs - 1
                kept = nz & (r < keep)
                if plus_one:
                    obuf[o] = jnp.where(kept, r + 1, 0)
                else:
                    obuf[o] = jnp.where(kept, r, -1)
                return run + jnp.sum(nzi)

            run = lax.fori_loop(0, 128, inner, run)
            pltpu.sync_copy(obuf, out_hbm.at[pl.ds(off, 2048)])
            return run

        lax.fori_loop(0, nchunk, outer, pfx)

    def run(tab, cnt):
        k = pl.kernel(
            body,
            out_type=jax.ShapeDtypeStruct((total_words,), jnp.int32),
            mesh=_mesh(),
        compiler_params=pltpu.CompilerParams(needs_layout_passes=False),
            scratch_types=[
                pltpu.VMEM((2048,), jnp.int32),
                pltpu.VMEM((2048,), jnp.int32),
                pltpu.VMEM((16, 16), jnp.int32),
            ],
        )
        return k(tab, cnt)

    return run


_run_k4 = _make_compact(P1S, P1W, NB_REAL, NSLOTS, False)
_run_k7 = _make_compact(T2S, T2W, T2REAL, MAXV, True)


def _k5(ids_hbm, map_hbm, t2_hbm, fidx_hbm, zb, idsv, bkt, mapg, fidxv, valv, sem):
    w = lax.axis_index("s")
    _zero_vec(zb, 2048)
    base2 = w * T2W

    def clr(c, _):
        off = pl.multiple_of(base2 + c * 2048, 8)
        pltpu.sync_copy(zb, t2_hbm.at[pl.ds(off, 2048)])
        return 0

    lax.fori_loop(0, T2W // 2048, clr, 0)
    plsc.subcore_barrier()
    pltpu.sync_copy(ids_hbm.at[w], idsv)

    def outer(j, _):
        def inner(k, _):
            o = pl.multiple_of(j * 128 + k * 16, 16)
            v = idsv[pl.ds(o, 16)]
            bkt[j, pl.ds(pl.multiple_of(k * 16, 16), 16)] = v >> 6
            return 0

        lax.fori_loop(0, 8, inner, 0)
        return 0

    lax.fori_loop(0, NROW, outer, 0)
    _fire_drain(NROW, lambda j: pltpu.make_async_copy(
        map_hbm.at[bkt.at[j]], mapg.at[j], sem))

    def outer2(j, _):
        def inner(k, _):
            o = pl.multiple_of(j * 128 + k * 16, 16)
            ko = pl.ds(pl.multiple_of(k * 16, 16), 16)
            v = idsv[pl.ds(o, 16)]
            s = mapg[j, ko]
            pos = w * PPW + j * 128 + k * 16 + _iota16()
            fidxv[j, ko] = jnp.where(s >= 0, s * 64 + (v & 63), T2REAL + pos)
            valv[j, ko] = v + 1
            return 0

        lax.fori_loop(0, 8, inner, 0)
        return 0

    lax.fori_loop(0, NROW, outer2, 0)
    _fire_drain(NROW, lambda j: pltpu.make_async_copy(
        valv.at[j], t2_hbm.at[fidxv.at[j]], sem))
    pltpu.sync_copy(fidxv, fidx_hbm.at[w])


def _run_k5(ids2d, mp):
    k = pl.kernel(
        _k5,
        out_type=(jax.ShapeDtypeStruct((T2S,), jnp.int32),
                  jax.ShapeDtypeStruct((NW, NROW, 128), jnp.int32)),
        mesh=_mesh(),
        compiler_params=pltpu.CompilerParams(needs_layout_passes=False),
        scratch_types=[
            pltpu.VMEM((2048,), jnp.int32),
            pltpu.VMEM((PPW,), jnp.int32),
            pltpu.VMEM((NROW, 128), jnp.int32),
            pltpu.VMEM((NROW, 128), jnp.int32),
            pltpu.VMEM((NROW, 128), jnp.int32),
            pltpu.VMEM((NROW, 128), jnp.int32),
            pltpu.SemaphoreType.DMA,
        ],
    )
    return k(ids2d, mp)


def _k8(fidx_hbm, rows_hbm, packed_hbm, counts_hbm, fidxv, rowsg, cnt_tbl,
        tmp, pkv, sem):
    w = lax.axis_index("s")
    it = _iota16()
    _zero_vec(cnt_tbl, CTW)
    pltpu.sync_copy(fidx_hbm.at[w], fidxv)
    _fire_drain(NROW, lambda j: pltpu.make_async_copy(
        rows_hbm.at[fidxv.at[j]], rowsg.at[j], sem))

    def outer(j, _):
        def inner(k, _):
            ko = pl.ds(pl.multiple_of(k * 16, 16), 16)
            r1 = rowsg[j, ko]
            rt = jnp.where(r1 > 0, r1 - 1, MAXV)
            comp = rt * 16 + it
            sk, _sv = plsc.sort_key_val(comp, comp)
            srow = sk >> 4
            slane = sk & 15
            tmp[...] = srow
            prev = plsc.load_gather(tmp, [jnp.maximum(it - 1, 0)])
            nxt = plsc.load_gather(tmp, [jnp.minimum(it + 1, 15)])
            new_run = (it == 0) | (srow != prev)
            is_last = (it == 15) | (srow != nxt)
            segst = plsc.cummax(jnp.where(new_run, it, 0))
            occ = it - segst
            old = plsc.load_gather(cnt_tbl, [srow])
            lr = old + occ
            plsc.store_scatter(cnt_tbl, [srow], lr + 1, mask=is_last)
            pk = (srow << 14) | lr
            pos = j * 128 + k * 16 + slane
            plsc.store_scatter(pkv, [pos], pk)
            return 0

        lax.fori_loop(0, 8, inner, 0)
        return 0

    lax.fori_loop(0, NROW, outer, 0)
    pltpu.sync_copy(pkv, packed_hbm.at[pl.ds(pl.multiple_of(w * PPW, 8), PPW)])
    pltpu.sync_copy(cnt_tbl, counts_hbm.at[w])


def _run_k8(fidx, t2rows):
    k = pl.kernel(
        _k8,
        out_type=(jax.ShapeDtypeStruct((NW * PPW,), jnp.int32),
                  jax.ShapeDtypeStruct((NW, CTW), jnp.int32)),
        mesh=_mesh(),
        compiler_params=pltpu.CompilerParams(needs_layout_passes=False),
        scratch_types=[
            pltpu.VMEM((NROW, 128), jnp.int32),
            pltpu.VMEM((NROW, 128), jnp.int32),
            pltpu.VMEM((CTW,), jnp.int32),
            pltpu.VMEM((16,), jnp.int32),
            pltpu.VMEM((PPW,), jnp.int32),
            pltpu.SemaphoreType.DMA,
        ],
    )
    return k(fidx, t2rows)


def _k9(counts_hbm, prefix_hbm, nump_hbm, accv, cb):
    w = lax.axis_index("s")
    sb = w * SLAB
    _zero_vec(accv, SLAB)

    for v in range(NW):
        off = pl.multiple_of(v * CTW + sb, 8)
        pltpu.sync_copy(accv, prefix_hbm.at[pl.ds(off, SLAB)])
        pltpu.sync_copy(counts_hbm.at[v].at[pl.ds(pl.multiple_of(sb, 8), SLAB)], cb)

        def add(i, _):
            o = pl.ds(pl.multiple_of(i * 16, 16), 16)
            accv[o] = accv[o] + cb[o]
            return 0

        lax.fori_loop(0, SLAB // 16, add, 0)

    def fin(i, _):
        o = pl.ds(pl.multiple_of(i * 16, 16), 16)
        cb[o] = jnp.minimum(accv[o], MAXP)
        return 0

    lax.fori_loop(0, SLAB // 16, fin, 0)
    pltpu.sync_copy(cb, nump_hbm.at[pl.ds(pl.multiple_of(sb, 8), SLAB)])


def _run_k9(counts):
    k = pl.kernel(
        _k9,
        out_type=(jax.ShapeDtypeStruct((NW * CTW,), jnp.int32),
                  jax.ShapeDtypeStruct((CTW,), jnp.int32)),
        mesh=_mesh(),
        compiler_params=pltpu.CompilerParams(needs_layout_passes=False),
        scratch_types=[
            pltpu.VMEM((SLAB,), jnp.int32),
            pltpu.VMEM((SLAB,), jnp.int32),
        ],
    )
    return k(counts)


def _k10(pts_hbm, pcrd_hbm, packed_hbm, prefix_hbm, vox_hbm, coors_hbm,
         zbf, zbi, pc0, pc1, pc2, pc3, idsv, pkv, pgi, pg,
         vi0, vi1, vi2, vi3, ci0, ci1, ci2, cv0, cv1, cv2, sem):
    w = lax.axis_index("s")
    it = _iota16()
    _zero_vec(zbf, 2048)
    _zero_vec(zbi, 2048)
    for c in range(35):
        pltpu.sync_copy(zbf, vox_hbm.at[pl.ds(
            pl.multiple_of(w * 71680 + c * 2048, 8), 2048)])
    for c in range(20):
        pltpu.sync_copy(zbi, coors_hbm.at[pl.ds(
            pl.multiple_of(w * 40960 + c * 2048, 8), 2048)])
    plsc.subcore_barrier()

    pcs = (pc0, pc1, pc2, pc3)
    vis = (vi0, vi1, vi2, vi3)
    cis = (ci0, ci1, ci2)
    cvs = (cv0, cv1, cv2)
    for h in range(2):
        base = w * PPW + h * HALF
        off = pl.ds(pl.multiple_of(base, 8), HALF)
        for c in range(4):
            pltpu.sync_copy(pts_hbm.at[c].at[off], pcs[c])
        pltpu.sync_copy(
            pcrd_hbm.at[w].at[pl.ds(pl.multiple_of(h * HALF, 8), HALF)], idsv)
        pltpu.sync_copy(packed_hbm.at[off], pkv)

        def bld1(j, _):
            def inner(k, _):
                o = pl.ds(pl.multiple_of(j * 128 + k * 16, 16), 16)
                ko = pl.ds(pl.multiple_of(k * 16, 16), 16)
                pk = pkv[o]
                pgi[j, ko] = w * CTW + (pk >> 14)
                return 0

            lax.fori_loop(0, 8, inner, 0)
            return 0

        lax.fori_loop(0, HROW, bld1, 0)
        _fire_drain(HROW, lambda j: pltpu.make_async_copy(
            prefix_hbm.at[pgi.at[j]], pg.at[j], sem))

        def bld2(j, _):
            def inner(k, _):
                o = pl.ds(pl.multiple_of(j * 128 + k * 16, 16), 16)
                ko = pl.ds(pl.multiple_of(k * 16, 16), 16)
                pk = pkv[o]
                rt = pk >> 14
                g = pg[j, ko] + (pk & 16383)
                okr = rt < MAXV
                keepv = okr & (g < MAXP)
                pos = w * PPW + h * HALF + j * 128 + k * 16 + it
                vf = jnp.where(keepv, (rt * 5 + g) * 4, VDUMPW + pos * 4)
                vi0[j, ko] = vf
                vi1[j, ko] = vf + 1
                vi2[j, ko] = vf + 2
                vi3[j, ko] = vf + 3
                cf = jnp.where(okr, rt * 3, CDUMPW + pos * 3)
                ci0[j, ko] = cf
                ci1[j, ko] = cf + 1
                ci2[j, ko] = cf + 2
                pcv = idsv[o]
                cv0[o] = pcv >> 22
                cv1[o] = (pcv >> 11) & 2047
                cv2[o] = pcv & 2047
                return 0

            lax.fori_loop(0, 8, inner, 0)
            return 0

        lax.fori_loop(0, HROW, bld2, 0)
        for c in range(4):
            vic = vis[c]
            pcc = pcs[c]
            _fire_drain(HROW, lambda j: pltpu.make_async_copy(
                pcc.at[pl.ds(j * 128, 128)], vox_hbm.at[vic.at[j]], sem))
        for c in range(3):
            cic = cis[c]
            cvc = cvs[c]
            _fire_drain(HROW, lambda j: pltpu.make_async_copy(
                cvc.at[pl.ds(j * 128, 128)], coors_hbm.at[cic.at[j]], sem))


def _run_k10(pts_t, pc2d, packed, prefix):
    k = pl.kernel(
        _k10,
        out_type=(jax.ShapeDtypeStruct((VWORDS,), jnp.float32),
                  jax.ShapeDtypeStruct((CWORDS,), jnp.int32)),
        mesh=_mesh(),
        compiler_params=pltpu.CompilerParams(needs_layout_passes=False),
        scratch_types=(
            [pltpu.VMEM((2048,), jnp.float32),
             pltpu.VMEM((2048,), jnp.int32)]
            + [pltpu.VMEM((HALF,), jnp.float32) for _ in range(4)]
            + [pltpu.VMEM((HALF,), jnp.int32),
               pltpu.VMEM((HALF,), jnp.int32)]
            + [pltpu.VMEM((HROW, 128), jnp.int32) for _ in range(2)]
            + [pltpu.VMEM((HROW, 128), jnp.int32) for _ in range(7)]
            + [pltpu.VMEM((HALF,), jnp.int32) for _ in range(3)]
            + [pltpu.SemaphoreType.DMA]
        ),
    )
    return k(pts_t, pc2d, packed, prefix)


def kernel(input):
    points = input
    ids2d, pc2d = _compute_ids(points)
    p1 = _run_k2(ids2d)
    cnt = _run_k3(p1)
    mp = _run_k4(p1, cnt)
    t2, fidx = _run_k5(ids2d, mp)
    fcnt = _run_k6(t2)
    t2rows = _run_k7(t2, fcnt)
    packed, counts = _run_k8(fidx, t2rows)
    prefix, nump_buf = _run_k9(counts)
    pts_t = jnp.pad(points, ((0, NPAD - N), (0, 0))).T.reshape(4, NPAD)
    vox_flat, coors_flat = _run_k10(pts_t, pc2d, packed, prefix)
    voxels = vox_flat[:MAXV * MAXP * 4].reshape(MAXV, MAXP, 4)
    coors = coors_flat[:MAXV * 3].reshape(MAXV, 3)
    return voxels, coors, nump_buf[:MAXV]


# speedup vs baseline: 140.3296x; 2.7289x over previous
import functools

import jax
import jax.numpy as jnp
from jax import lax
from jax.experimental import pallas as pl
from jax.experimental.pallas import tpu as pltpu
from jax.experimental.pallas import tpu_sc as plsc

GX, GY, GZ = 1408, 1600, 40
TOT = GX * GY * GZ
MAXV, MAXP = 16000, 5
N, NPAD = 200000, 200704
NW = 16
PPW = NPAD // NW
NROW = PPW // 128
P1S = 1409024
P1W = P1S // NW
NB_REAL = 1408000
NSLOTS = 16384
T2S = 1277952
T2W = T2S // NW
T2REAL = NSLOTS * 64
CTW = 16384
SLAB = CTW // NW
VWORDS, VDUMPW = 1146880, 320000
CWORDS, CDUMPW = 655360, 48384
HALF = PPW // 2
HROW = HALF // 128


def _mesh():
    return plsc.VectorSubcoreMesh(
        core_axis_name="c", subcore_axis_name="s", num_cores=1)


def _iota16():
    return lax.iota(jnp.int32, 16)


def _zero_vec(ref, nwords):
    z = jnp.zeros((16,), ref.dtype)

    def body(i, _):
        ref[pl.ds(pl.multiple_of(i * 16, 16), 16)] = z
        return 0

    lax.fori_loop(0, nwords // 16, body, 0)


def _fire_drain(n, mk, chunk=8):
    for c0 in range(0, n, chunk):
        descs = []
        for j in range(c0, min(n, c0 + chunk)):
            d = mk(j)
            d.start()
            descs.append(d)
        for d in descs:
            d.wait()


def _ids_body(x_ref, y_ref, z_ref, id_ref, pc_ref):
    x = x_ref[...]
    y = y_ref[...]
    z = z_ref[...]
    cx = jnp.floor((x - 0.0) / jnp.float32(0.05)).astype(jnp.int32)
    cy = jnp.floor((y - jnp.float32(-40.0)) / jnp.float32(0.05)).astype(jnp.int32)
    cz = jnp.floor((z - jnp.float32(-3.0)) / jnp.float32(0.1)).astype(jnp.int32)
    ok = (cx >= 0) & (cx < GX) & (cy >= 0) & (cy < GY) & (cz >= 0) & (cz < GZ)
    id_ref[...] = jnp.where(ok, (cz * GY + cy) * GX + cx, TOT)
    pc_ref[...] = (cz << 22) | (cy << 11) | cx


def _compute_ids(points):
    pad = NPAD - N
    x = jnp.pad(points[:, 0], (0, pad), constant_values=-1.0).reshape(1568, 128)
    y = jnp.pad(points[:, 1], (0, pad)).reshape(1568, 128)
    z = jnp.pad(points[:, 2], (0, pad)).reshape(1568, 128)
    ids, pc = pl.pallas_call(
        _ids_body,
        out_shape=(jax.ShapeDtypeStruct((1568, 128), jnp.int32),
                   jax.ShapeDtypeStruct((1568, 128), jnp.int32)),
    )(x, y, z)
    return ids.reshape(NW, PPW), pc.reshape(NW, PPW)


def _k2(ids_hbm, p1_hbm, zb, idsv, bkt, ones, sem):
    w = lax.axis_index("s")
    _zero_vec(zb, 2048)
    one = jnp.ones((16,), jnp.int32)
    for k in range(8):
        ones[pl.ds(k * 16, 16)] = one

    base = w * P1W

    def clr(c, _):
        off = pl.multiple_of(base + c * 2048, 8)
        pltpu.sync_copy(zb, p1_hbm.at[pl.ds(off, 2048)])
        return 0

    lax.fori_loop(0, P1W // 2048, clr, 0)
    plsc.subcore_barrier()
    pltpu.sync_copy(ids_hbm.at[w], idsv)

    def outer(j, _):
        def inner(k, _):
            o = pl.multiple_of(j * 128 + k * 16, 16)
            v = idsv[pl.ds(o, 16)]
            bkt[j, pl.ds(pl.multiple_of(k * 16, 16), 16)] = v >> 6
            return 0

        lax.fori_loop(0, 8, inner, 0)
        return 0

    lax.fori_loop(0, NROW, outer, 0)

    def mk(j):
        return pltpu.make_async_copy(ones, p1_hbm.at[bkt.at[j]], sem)

    _fire_drain(NROW, mk)


def _run_k2(ids2d):
    k = pl.kernel(
        _k2,
        out_type=jax.ShapeDtypeStruct((P1S,), jnp.int32),
        mesh=_mesh(),
        compiler_params=pltpu.CompilerParams(needs_layout_passes=False),
        scratch_types=[
            pltpu.VMEM((2048,), jnp.int32),
            pltpu.VMEM((PPW,), jnp.int32),
            pltpu.VMEM((NROW, 128), jnp.int32),
            pltpu.VMEM((128,), jnp.int32),
            pltpu.SemaphoreType.DMA,
        ],
    )
    return k(ids2d)


def _make_count(total_words, per_w, real_limit):
    nchunk = per_w // 2048

    def body(tab_hbm, cnt_hbm, buf, stg):
        w = lax.axis_index("s")
        base = w * per_w
        it = _iota16()

        def outer(c, acc):
            off = pl.multiple_of(base + c * 2048, 8)
            pltpu.sync_copy(tab_hbm.at[pl.ds(off, 2048)], buf)
            gbase = base + c * 2048

            def inner(i, acc):
                v = buf[pl.ds(pl.multiple_of(i * 16, 16), 16)]
                g = gbase + i * 16 + it
                nz = (v != 0) & (g < real_limit)
                return acc + jnp.where(nz, 1, 0)

            return lax.fori_loop(0, 128, inner, acc)

        acc = lax.fori_loop(0, nchunk, outer, jnp.zeros((16,), jnp.int32))
        stg[...] = jnp.broadcast_to(jnp.sum(acc), (16,))
        pltpu.sync_copy(stg, cnt_hbm.at[w])

    def run(tab):
        k = pl.kernel(
            body,
            out_type=jax.ShapeDtypeStruct((16, 16), jnp.int32),
            mesh=_mesh(),
        compiler_params=pltpu.CompilerParams(needs_layout_passes=False),
            scratch_types=[
                pltpu.VMEM((2048,), jnp.int32),
                pltpu.VMEM((16,), jnp.int32),
            ],
        )
        return k(tab)

    return run


_run_k3 = _make_count(P1S, P1W, NB_REAL)
_run_k6 = _make_count(T2S, T2W, T2REAL)


def _worker_prefix(cnt_hbm, cntv, w):
    it = _iota16()
    pltpu.sync_copy(cnt_hbm, cntv)
    totals = plsc.load_gather(cntv, [it, jnp.zeros((16,), jnp.int32)])
    return jnp.sum(jnp.where(it < w, totals, 0))


def _make_compact(total_words, per_w, real_limit, keep, plus_one):
    nchunk = per_w // 2048

    def body(tab_hbm, cnt_hbm, out_hbm, buf, obuf, cntv):
        w = lax.axis_index("s")
        base = w * per_w
        it = _iota16()
        pfx = _worker_prefix(cnt_hbm, cntv, w)

        def outer(c, run):
            off = pl.multiple_of(base + c * 2048, 8)
            pltpu.sync_copy(tab_hbm.at[pl.ds(off, 2048)], buf)
            gbase = base + c * 2048

            def inner(i, run):
                o = pl.ds(pl.multiple_of(i * 16, 16), 16)
                v = buf[o]
                g = gbase + i * 16 + it
                nz = (v != 0) & (g < real_limit)
                nzi = nz.astype(jnp.int32)
                cs = plsc.cumsum(nzi)
                r = run + cs - 1
                kept = nz & (r < keep)
                if plus_one:
                    obuf[o] = jnp.where(kept, r + 1, 0)
                else:
                    obuf[o] = jnp.where(kept, r, -1)
                return run + jnp.sum(nzi)

            run = lax.fori_loop(0, 128, inner, run)
            pltpu.sync_copy(obuf, out_hbm.at[pl.ds(off, 2048)])
            return run

        lax.fori_loop(0, nchunk, outer, pfx)

    def run(tab, cnt):
        k = pl.kernel(
            body,
            out_type=jax.ShapeDtypeStruct((total_words,), jnp.int32),
            mesh=_mesh(),
        compiler_params=pltpu.CompilerParams(needs_layout_passes=False),
            scratch_types=[
                pltpu.VMEM((2048,), jnp.int32),
                pltpu.VMEM((2048,), jnp.int32),
                pltpu.VMEM((16, 16), jnp.int32),
            ],
        )
        return k(tab, cnt)

    return run


_run_k4 = _make_compact(P1S, P1W, NB_REAL, NSLOTS, False)
_run_k7 = _make_compact(T2S, T2W, T2REAL, MAXV, True)


def _k5(ids_hbm, map_hbm, t2_hbm, fidx_hbm, zb, idsv, bkt, mapg, fidxv, valv, sem):
    w = lax.axis_index("s")
    _zero_vec(zb, 2048)
    base2 = w * T2W

    def clr(c, _):
        off = pl.multiple_of(base2 + c * 2048, 8)
        pltpu.sync_copy(zb, t2_hbm.at[pl.ds(off, 2048)])
        return 0

    lax.fori_loop(0, T2W // 2048, clr, 0)
    plsc.subcore_barrier()
    pltpu.sync_copy(ids_hbm.at[w], idsv)

    def outer(j, _):
        def inner(k, _):
            o = pl.multiple_of(j * 128 + k * 16, 16)
            v = idsv[pl.ds(o, 16)]
            bkt[j, pl.ds(pl.multiple_of(k * 16, 16), 16)] = v >> 6
            return 0

        lax.fori_loop(0, 8, inner, 0)
        return 0

    lax.fori_loop(0, NROW, outer, 0)
    _fire_drain(NROW, lambda j: pltpu.make_async_copy(
        map_hbm.at[bkt.at[j]], mapg.at[j], sem))

    def outer2(j, _):
        def inner(k, _):
            o = pl.multiple_of(j * 128 + k * 16, 16)
            ko = pl.ds(pl.multiple_of(k * 16, 16), 16)
            v = idsv[pl.ds(o, 16)]
            s = mapg[j, ko]
            pos = w * PPW + j * 128 + k * 16 + _iota16()
            fidxv[j, ko] = jnp.where(s >= 0, s * 64 + (v & 63), T2REAL + pos)
            valv[j, ko] = v + 1
            return 0

        lax.fori_loop(0, 8, inner, 0)
        return 0

    lax.fori_loop(0, NROW, outer2, 0)
    _fire_drain(NROW, lambda j: pltpu.make_async_copy(
        valv.at[j], t2_hbm.at[fidxv.at[j]], sem))
    pltpu.sync_copy(fidxv, fidx_hbm.at[w])


def _run_k5(ids2d, mp):
    k = pl.kernel(
        _k5,
        out_type=(jax.ShapeDtypeStruct((T2S,), jnp.int32),
                  jax.ShapeDtypeStruct((NW, NROW, 128), jnp.int32)),
        mesh=_mesh(),
        compiler_params=pltpu.CompilerParams(needs_layout_passes=False),
        scratch_types=[
            pltpu.VMEM((2048,), jnp.int32),
            pltpu.VMEM((PPW,), jnp.int32),
            pltpu.VMEM((NROW, 128), jnp.int32),
            pltpu.VMEM((NROW, 128), jnp.int32),
            pltpu.VMEM((NROW, 128), jnp.int32),
            pltpu.VMEM((NROW, 128), jnp.int32),
            pltpu.SemaphoreType.DMA,
        ],
    )
    return k(ids2d, mp)


def _k8(fidx_hbm, rows_hbm, packed_hbm, counts_hbm, fidxv, rowsg, cnt_tbl,
        tmp, pkv, sem):
    w = lax.axis_index("s")
    it = _iota16()
    _zero_vec(cnt_tbl, CTW)
    pltpu.sync_copy(fidx_hbm.at[w], fidxv)
    _fire_drain(NROW, lambda j: pltpu.make_async_copy(
        rows_hbm.at[fidxv.at[j]], rowsg.at[j], sem))

    def outer(j, _):
        def inner(k, _):
            ko = pl.ds(pl.multiple_of(k * 16, 16), 16)
            r1 = rowsg[j, ko]
            rt = jnp.where(r1 > 0, r1 - 1, MAXV)
            comp = rt * 16 + it
            sk, _sv = plsc.sort_key_val(comp, comp)
            srow = sk >> 4
            slane = sk & 15
            tmp[...] = srow
            prev = plsc.load_gather(tmp, [jnp.maximum(it - 1, 0)])
            nxt = plsc.load_gather(tmp, [jnp.minimum(it + 1, 15)])
            new_run = (it == 0) | (srow != prev)
            is_last = (it == 15) | (srow != nxt)
            segst = plsc.cummax(jnp.where(new_run, it, 0))
            occ = it - segst
            old = plsc.load_gather(cnt_tbl, [srow])
            lr = old + occ
            plsc.store_scatter(cnt_tbl, [srow], lr + 1, mask=is_last)
            pk = (srow << 14) | lr
            pos = j * 128 + k * 16 + slane
            plsc.store_scatter(pkv, [pos], pk)
            return 0

        lax.fori_loop(0, 8, inner, 0)
        return 0

    lax.fori_loop(0, NROW, outer, 0)
    pltpu.sync_copy(pkv, packed_hbm.at[pl.ds(pl.multiple_of(w * PPW, 8), PPW)])
    pltpu.sync_copy(cnt_tbl, counts_hbm.at[w])


def _run_k8(fidx, t2rows):
    k = pl.kernel(
        _k8,
        out_type=(jax.ShapeDtypeStruct((NW * PPW,), jnp.int32),
                  jax.ShapeDtypeStruct((NW, CTW), jnp.int32)),
        mesh=_mesh(),
        compiler_params=pltpu.CompilerParams(needs_layout_passes=False),
        scratch_types=[
            pltpu.VMEM((NROW, 128), jnp.int32),
            pltpu.VMEM((NROW, 128), jnp.int32),
            pltpu.VMEM((CTW,), jnp.int32),
            pltpu.VMEM((16,), jnp.int32),
            pltpu.VMEM((PPW,), jnp.int32),
            pltpu.SemaphoreType.DMA,
        ],
    )
    return k(fidx, t2rows)


def _k9(counts_hbm, prefix_hbm, nump_hbm, accv, cb):
    w = lax.axis_index("s")
    sb = w * SLAB
    _zero_vec(accv, SLAB)

    for v in range(NW):
        off = pl.multiple_of(v * CTW + sb, 8)
        pltpu.sync_copy(accv, prefix_hbm.at[pl.ds(off, SLAB)])
        pltpu.sync_copy(counts_hbm.at[v].at[pl.ds(pl.multiple_of(sb, 8), SLAB)], cb)

        def add(i, _):
            o = pl.ds(pl.multiple_of(i * 16, 16), 16)
            accv[o] = accv[o] + cb[o]
            return 0

        lax.fori_loop(0, SLAB // 16, add, 0)

    def fin(i, _):
        o = pl.ds(pl.multiple_of(i * 16, 16), 16)
        cb[o] = jnp.minimum(accv[o], MAXP)
        return 0

    lax.fori_loop(0, SLAB // 16, fin, 0)
    pltpu.sync_copy(cb, nump_hbm.at[pl.ds(pl.multiple_of(sb, 8), SLAB)])


def _run_k9(counts):
    k = pl.kernel(
        _k9,
        out_type=(jax.ShapeDtypeStruct((NW * CTW,), jnp.int32),
                  jax.ShapeDtypeStruct((CTW,), jnp.int32)),
        mesh=_mesh(),
        compiler_params=pltpu.CompilerParams(needs_layout_passes=False),
        scratch_types=[
            pltpu.VMEM((SLAB,), jnp.int32),
            pltpu.VMEM((SLAB,), jnp.int32),
        ],
    )
    return k(counts)


def _k10(px_hbm, py_hbm, pz_hbm, pw_hbm, pcrd_hbm, packed_hbm, prefix_hbm,
         vox_hbm, coors_hbm,
         zbf, zbi, pkv, cand, pidx, pgi, pg,
         vi0, vi1, vi2, vi3, ci0, ci1, ci2,
         px0, px1, px2, px3, cv0, cv1, cv2, sem):
    w = lax.axis_index("s")
    it = _iota16()
    _zero_vec(zbf, 2048)
    _zero_vec(zbi, 2048)
    for c in range(35):
        pltpu.sync_copy(zbf, vox_hbm.at[pl.ds(
            pl.multiple_of(w * 71680 + c * 2048, 8), 2048)])
    for c in range(20):
        pltpu.sync_copy(zbi, coors_hbm.at[pl.ds(
            pl.multiple_of(w * 40960 + c * 2048, 8), 2048)])
    plsc.subcore_barrier()

    for (r0, nr) in ((0, 24), (24, 24), (48, 24), (72, 26)):
        base = w * PPW + r0 * 128
        npts = nr * 128
        pltpu.sync_copy(packed_hbm.at[pl.ds(pl.multiple_of(base, 8), npts)],
                        pkv.at[pl.ds(0, npts)])

        def pass_a(i, cnt):
            o = pl.ds(pl.multiple_of(i * 16, 16), 16)
            pk = pkv[o]
            rt = pk >> 14
            okm = rt < MAXV
            oki = okm.astype(jnp.int32)
            tgt = cnt + plsc.cumsum(oki) - 1
            pos16 = i * 16 + it
            plsc.store_scatter(cand, [jnp.where(okm, tgt, 3455)], pos16,
                               mask=okm)
            return cnt + jnp.sum(oki)

        cnt = lax.fori_loop(0, nr * 8, pass_a, jnp.int32(0))
        nch = (cnt + 127) >> 7

        def pass_b(j, _):
            for k in range(8):
                ko = pl.ds(pl.multiple_of(k * 16, 16), 16)
                slot = j * 128 + k * 16 + it
                msk = slot < cnt
                cl = plsc.load_gather(cand, [jnp.where(msk, slot, 0)])
                cpos = jnp.where(msk, cl, 0)
                pk = plsc.load_gather(pkv, [cpos])
                rt = pk >> 14
                pgi[j, ko] = jnp.where(msk, w * CTW + rt, w * CTW)
                pidx[j, ko] = jnp.where(msk, base + cpos, base)
            pltpu.make_async_copy(
                prefix_hbm.at[pgi.at[j]], pg.at[j], sem).start()
            pltpu.make_async_copy(
                prefix_hbm.at[pgi.at[j]], pg.at[j], sem).wait()
            return 0

        lax.fori_loop(0, nch, pass_b, 0)

        def gfeat(j, _):
            srcs = ((px_hbm, px0), (py_hbm, px1), (pz_hbm, px2), (pw_hbm, px3))
            for srcr, dst in srcs:
                pltpu.make_async_copy(
                    srcr.at[pidx.at[j]], dst.at[j], sem).start()
            pltpu.make_async_copy(
                pcrd_hbm.at[pidx.at[j]], cv0.at[j], sem).start()
            for srcr, dst in srcs:
                pltpu.make_async_copy(
                    srcr.at[pidx.at[j]], dst.at[j], sem).wait()
            pltpu.make_async_copy(
                pcrd_hbm.at[pidx.at[j]], cv0.at[j], sem).wait()
            return 0

        lax.fori_loop(0, nch, gfeat, 0)

        def pass_c(j, _):
            for k in range(8):
                ko = pl.ds(pl.multiple_of(k * 16, 16), 16)
                slot = j * 128 + k * 16 + it
                msk = slot < cnt
                cl = plsc.load_gather(cand, [jnp.where(msk, slot, 0)])
                cpos = jnp.where(msk, cl, 0)
                pk = plsc.load_gather(pkv, [cpos])
                rt = pk >> 14
                lr = pk & 16383
                g = pg[j, ko] + lr
                keepv = msk & (g < MAXP)
                dp = base + j * 128 + k * 16 + it
                vf = jnp.where(keepv, (rt * 5 + g) * 4, VDUMPW + dp * 4)
                vi0[j, ko] = vf
                vi1[j, ko] = vf + 1
                vi2[j, ko] = vf + 2
                vi3[j, ko] = vf + 3
                cf = jnp.where(msk, rt * 3, CDUMPW + dp * 3)
                ci0[j, ko] = cf
                ci1[j, ko] = cf + 1
                ci2[j, ko] = cf + 2
                pcv = cv0[j, ko]
                cv1[j, ko] = (pcv >> 11) & 2047
                cv2[j, ko] = pcv & 2047
                cv0[j, ko] = pcv >> 22
            return 0

        lax.fori_loop(0, nch, pass_c, 0)

        def scat(j, _):
            pairs = ((px0, vi0), (px1, vi1), (px2, vi2), (px3, vi3))
            cpairs = ((cv0, ci0), (cv1, ci1), (cv2, ci2))
            for srcb, idxb in pairs:
                pltpu.make_async_copy(
                    srcb.at[j], vox_hbm.at[idxb.at[j]], sem).start()
            for srcb, idxb in cpairs:
                pltpu.make_async_copy(
                    srcb.at[j], coors_hbm.at[idxb.at[j]], sem).start()
            for srcb, idxb in pairs:
                pltpu.make_async_copy(
                    srcb.at[j], vox_hbm.at[idxb.at[j]], sem).wait()
            for srcb, idxb in cpairs:
                pltpu.make_async_copy(
                    srcb.at[j], coors_hbm.at[idxb.at[j]], sem).wait()
            return 0

        lax.fori_loop(0, nch, scat, 0)


def _run_k10(px, py, pz, pw, pcf, packed, prefix):
    nb = 26
    k = pl.kernel(
        _k10,
        out_type=(jax.ShapeDtypeStruct((VWORDS,), jnp.float32),
                  jax.ShapeDtypeStruct((CWORDS,), jnp.int32)),
        mesh=_mesh(),
        compiler_params=pltpu.CompilerParams(needs_layout_passes=False),
        scratch_types=(
            [pltpu.VMEM((2048,), jnp.float32),
             pltpu.VMEM((2048,), jnp.int32),
             pltpu.VMEM((3328,), jnp.int32),
             pltpu.VMEM((3456,), jnp.int32)]
            + [pltpu.VMEM((nb, 128), jnp.int32) for _ in range(3)]
            + [pltpu.VMEM((nb, 128), jnp.int32) for _ in range(7)]
            + [pltpu.VMEM((nb, 128), jnp.float32) for _ in range(4)]
            + [pltpu.VMEM((nb, 128), jnp.int32) for _ in range(3)]
            + [pltpu.SemaphoreType.DMA]
        ),
    )
    return k(px, py, pz, pw, pcf, packed, prefix)


def kernel(input):
    points = input
    ids2d, pc2d = _compute_ids(points)
    p1 = _run_k2(ids2d)
    cnt = _run_k3(p1)
    mp = _run_k4(p1, cnt)
    t2, fidx = _run_k5(ids2d, mp)
    fcnt = _run_k6(t2)
    t2rows = _run_k7(t2, fcnt)
    packed, counts = _run_k8(fidx, t2rows)
    prefix, nump_buf = _run_k9(counts)
    pp = jnp.pad(points, ((0, NPAD - N), (0, 0)))
    vox_flat, coors_flat = _run_k10(
        pp[:, 0], pp[:, 1], pp[:, 2], pp[:, 3], pc2d.reshape(NPAD),
        packed, prefix)
    voxels = vox_flat[:MAXV * MAXP * 4].reshape(MAXV, MAXP, 4)
    coors = coors_flat[:MAXV * 3].reshape(MAXV, 3)
    return voxels, coors, nump_buf[:MAXV]


# speedup vs baseline: 210.2629x; 1.4984x over previous
import functools

import jax
import jax.numpy as jnp
from jax import lax
from jax.experimental import pallas as pl
from jax.experimental.pallas import tpu as pltpu
from jax.experimental.pallas import tpu_sc as plsc

GX, GY, GZ = 1408, 1600, 40
TOT = GX * GY * GZ
MAXV, MAXP = 16000, 5
N, NPAD = 200000, 200704
NW = 16
PPW = NPAD // NW
NROW = PPW // 128
P1S = 1409024
P1W = P1S // NW
NB_REAL = 1408000
NSLOTS = 16384
T2S = 1277952
T2W = T2S // NW
T2REAL = NSLOTS * 64
CTW = 16384
SLAB = CTW // NW
VWORDS, VDUMPW = 1146880, 320000
CWORDS, CDUMPW = 655360, 48384
HALF = PPW // 2
HROW = HALF // 128


def _mesh():
    return plsc.VectorSubcoreMesh(
        core_axis_name="c", subcore_axis_name="s", num_cores=1)


def _iota16():
    return lax.iota(jnp.int32, 16)


def _zero_vec(ref, nwords):
    z = jnp.zeros((16,), ref.dtype)

    def body(i, _):
        ref[pl.ds(pl.multiple_of(i * 16, 16), 16)] = z
        return 0

    lax.fori_loop(0, nwords // 16, body, 0)


def _fire_drain(n, mk, chunk=8):
    for c0 in range(0, n, chunk):
        descs = []
        for j in range(c0, min(n, c0 + chunk)):
            d = mk(j)
            d.start()
            descs.append(d)
        for d in descs:
            d.wait()


def _ids_body(x_ref, y_ref, z_ref, id_ref, pc_ref):
    x = x_ref[...]
    y = y_ref[...]
    z = z_ref[...]
    cx = jnp.floor((x - 0.0) / jnp.float32(0.05)).astype(jnp.int32)
    cy = jnp.floor((y - jnp.float32(-40.0)) / jnp.float32(0.05)).astype(jnp.int32)
    cz = jnp.floor((z - jnp.float32(-3.0)) / jnp.float32(0.1)).astype(jnp.int32)
    ok = (cx >= 0) & (cx < GX) & (cy >= 0) & (cy < GY) & (cz >= 0) & (cz < GZ)
    id_ref[...] = jnp.where(ok, (cz * GY + cy) * GX + cx, TOT)
    pc_ref[...] = (cz << 22) | (cy << 11) | cx


def _compute_ids(points):
    pad = NPAD - N
    x = jnp.pad(points[:, 0], (0, pad), constant_values=-1.0).reshape(1568, 128)
    y = jnp.pad(points[:, 1], (0, pad)).reshape(1568, 128)
    z = jnp.pad(points[:, 2], (0, pad)).reshape(1568, 128)
    ids, pc = pl.pallas_call(
        _ids_body,
        out_shape=(jax.ShapeDtypeStruct((1568, 128), jnp.int32),
                   jax.ShapeDtypeStruct((1568, 128), jnp.int32)),
    )(x, y, z)
    return ids.reshape(NW, PPW), pc.reshape(NW, PPW)


def _k2(ids_hbm, p1_hbm, zb, idsv, bkt, ones, sem):
    w = lax.axis_index("s")
    _zero_vec(zb, 2048)
    one = jnp.ones((16,), jnp.int32)
    for k in range(8):
        ones[pl.ds(k * 16, 16)] = one

    base = w * P1W

    def clr(c, _):
        off = pl.multiple_of(base + c * 2048, 8)
        pltpu.sync_copy(zb, p1_hbm.at[pl.ds(off, 2048)])
        return 0

    lax.fori_loop(0, P1W // 2048, clr, 0)
    plsc.subcore_barrier()
    pltpu.sync_copy(ids_hbm.at[w], idsv)

    def outer(j, _):
        def inner(k, _):
            o = pl.multiple_of(j * 128 + k * 16, 16)
            v = idsv[pl.ds(o, 16)]
            bkt[j, pl.ds(pl.multiple_of(k * 16, 16), 16)] = v >> 6
            return 0

        lax.fori_loop(0, 8, inner, 0)
        return 0

    lax.fori_loop(0, NROW, outer, 0)

    def mk(j):
        return pltpu.make_async_copy(ones, p1_hbm.at[bkt.at[j]], sem)

    _fire_drain(NROW, mk)


def _run_k2(ids2d):
    k = pl.kernel(
        _k2,
        out_type=jax.ShapeDtypeStruct((P1S,), jnp.int32),
        mesh=_mesh(),
        compiler_params=pltpu.CompilerParams(needs_layout_passes=False),
        scratch_types=[
            pltpu.VMEM((2048,), jnp.int32),
            pltpu.VMEM((PPW,), jnp.int32),
            pltpu.VMEM((NROW, 128), jnp.int32),
            pltpu.VMEM((128,), jnp.int32),
            pltpu.SemaphoreType.DMA,
        ],
    )
    return k(ids2d)


def _make_count(total_words, per_w, real_limit):
    nchunk = per_w // 2048

    def body(tab_hbm, cnt_hbm, buf, stg):
        w = lax.axis_index("s")
        base = w * per_w
        it = _iota16()

        def outer(c, acc):
            off = pl.multiple_of(base + c * 2048, 8)
            pltpu.sync_copy(tab_hbm.at[pl.ds(off, 2048)], buf)
            gbase = base + c * 2048

            def inner(i, acc):
                v = buf[pl.ds(pl.multiple_of(i * 16, 16), 16)]
                g = gbase + i * 16 + it
                nz = (v != 0) & (g < real_limit)
                return acc + jnp.where(nz, 1, 0)

            return lax.fori_loop(0, 128, inner, acc)

        acc = lax.fori_loop(0, nchunk, outer, jnp.zeros((16,), jnp.int32))
        stg[...] = jnp.broadcast_to(jnp.sum(acc), (16,))
        pltpu.sync_copy(stg, cnt_hbm.at[w])

    def run(tab):
        k = pl.kernel(
            body,
            out_type=jax.ShapeDtypeStruct((16, 16), jnp.int32),
            mesh=_mesh(),
        compiler_params=pltpu.CompilerParams(needs_layout_passes=False),
            scratch_types=[
                pltpu.VMEM((2048,), jnp.int32),
                pltpu.VMEM((16,), jnp.int32),
            ],
        )
        return k(tab)

    return run


_run_k3 = _make_count(P1S, P1W, NB_REAL)
_run_k6 = _make_count(T2S, T2W, T2REAL)


def _worker_prefix(cnt_hbm, cntv, w):
    it = _iota16()
    pltpu.sync_copy(cnt_hbm, cntv)
    totals = plsc.load_gather(cntv, [it, jnp.zeros((16,), jnp.int32)])
    return jnp.sum(jnp.where(it < w, totals, 0))


def _make_compact(total_words, per_w, real_limit, keep, plus_one):
    nchunk = per_w // 2048

    def body(tab_hbm, cnt_hbm, out_hbm, buf, obuf, cntv):
        w = lax.axis_index("s")
        base = w * per_w
        it = _iota16()
        pfx = _worker_prefix(cnt_hbm, cntv, w)

        def outer(c, run):
            off = pl.multiple_of(base + c * 2048, 8)
            pltpu.sync_copy(tab_hbm.at[pl.ds(off, 2048)], buf)
            gbase = base + c * 2048

            def inner(i, run):
                o = pl.ds(pl.multiple_of(i * 16, 16), 16)
                v = buf[o]
                g = gbase + i * 16 + it
                nz = (v != 0) & (g < real_limit)
                nzi = nz.astype(jnp.int32)
                cs = plsc.cumsum(nzi)
                r = run + cs - 1
                kept = nz & (r < keep)
                if plus_one:
                    obuf[o] = jnp.where(kept, r + 1, 0)
                else:
                    obuf[o] = jnp.where(kept, r, -1)
                return run + jnp.sum(nzi)

            run = lax.fori_loop(0, 128, inner, run)
            pltpu.sync_copy(obuf, out_hbm.at[pl.ds(off, 2048)])
            return run

        lax.fori_loop(0, nchunk, outer, pfx)

    def run(tab, cnt):
        k = pl.kernel(
            body,
            out_type=jax.ShapeDtypeStruct((total_words,), jnp.int32),
            mesh=_mesh(),
        compiler_params=pltpu.CompilerParams(needs_layout_passes=False),
            scratch_types=[
                pltpu.VMEM((2048,), jnp.int32),
                pltpu.VMEM((2048,), jnp.int32),
                pltpu.VMEM((16, 16), jnp.int32),
            ],
        )
        return k(tab, cnt)

    return run


_run_k4 = _make_compact(P1S, P1W, NB_REAL, NSLOTS, False)
_run_k7 = _make_compact(T2S, T2W, T2REAL, MAXV, True)


def _k5(ids_hbm, map_hbm, t2_hbm, fidx_hbm, zb, idsv, bkt, mapg, fidxv,
        cand, fic, vc, sem):
    w = lax.axis_index("s")
    it = _iota16()
    _zero_vec(zb, 2048)
    base2 = w * (T2REAL // NW)

    def clr(c, _):
        off = pl.multiple_of(base2 + c * 2048, 8)
        pltpu.sync_copy(zb, t2_hbm.at[pl.ds(off, 2048)])
        return 0

    lax.fori_loop(0, T2REAL // NW // 2048, clr, 0)
    plsc.subcore_barrier()
    pltpu.sync_copy(ids_hbm.at[w], idsv)

    def outer(j, _):
        def inner(k, _):
            o = pl.multiple_of(j * 128 + k * 16, 16)
            v = idsv[pl.ds(o, 16)]
            bkt[j, pl.ds(pl.multiple_of(k * 16, 16), 16)] = v >> 6
            return 0

        lax.fori_loop(0, 8, inner, 0)
        return 0

    lax.fori_loop(0, NROW, outer, 0)
    _fire_drain(NROW, lambda j: pltpu.make_async_copy(
        map_hbm.at[bkt.at[j]], mapg.at[j], sem))

    def outer2(j, cnt):
        def inner(k, cnt):
            o = pl.multiple_of(j * 128 + k * 16, 16)
            ko = pl.ds(pl.multiple_of(k * 16, 16), 16)
            v = idsv[pl.ds(o, 16)]
            s = mapg[j, ko]
            pos = w * PPW + j * 128 + k * 16 + it
            fidxv[j, ko] = jnp.where(s >= 0, s * 64 + (v & 63), T2REAL + pos)
            okm = s >= 0
            oki = okm.astype(jnp.int32)
            tgt = cnt + plsc.cumsum(oki) - 1
            lpos = j * 128 + k * 16 + it
            plsc.store_scatter(cand, [jnp.where(okm, tgt, 12671)], lpos,
                               mask=okm)
            return cnt + jnp.sum(oki)

        return lax.fori_loop(0, 8, inner, cnt)

    cnt = lax.fori_loop(0, NROW, outer2, jnp.int32(0))
    pltpu.sync_copy(fidxv, fidx_hbm.at[w])
    nch = (cnt + 127) >> 7

    def pass_s(j, _):
        for k in range(8):
            ko = pl.ds(pl.multiple_of(k * 16, 16), 16)
            slot = j * 128 + k * 16 + it
            msk = slot < cnt
            cl = plsc.load_gather(cand, [jnp.where(msk, slot, 0)])
            cpos = jnp.where(msk, cl, 0)
            fx = plsc.load_gather(fidxv, [cpos >> 7, cpos & 127])
            vv = plsc.load_gather(idsv, [cpos])
            fic[j, ko] = jnp.where(msk, fx, T2REAL + w * PPW + slot)
            vc[j, ko] = vv + 1
        pltpu.make_async_copy(vc.at[j], t2_hbm.at[fic.at[j]], sem).start()
        pltpu.make_async_copy(vc.at[j], t2_hbm.at[fic.at[j]], sem).wait()
        return 0

    lax.fori_loop(0, nch, pass_s, 0)


def _run_k5(ids2d, mp):
    k = pl.kernel(
        _k5,
        out_type=(jax.ShapeDtypeStruct((T2S,), jnp.int32),
                  jax.ShapeDtypeStruct((NW, NROW, 128), jnp.int32)),
        mesh=_mesh(),
        compiler_params=pltpu.CompilerParams(needs_layout_passes=False),
        scratch_types=[
            pltpu.VMEM((2048,), jnp.int32),
            pltpu.VMEM((PPW,), jnp.int32),
            pltpu.VMEM((NROW, 128), jnp.int32),
            pltpu.VMEM((NROW, 128), jnp.int32),
            pltpu.VMEM((NROW, 128), jnp.int32),
            pltpu.VMEM((12672,), jnp.int32),
            pltpu.VMEM((NROW, 128), jnp.int32),
            pltpu.VMEM((NROW, 128), jnp.int32),
            pltpu.SemaphoreType.DMA,
        ],
    )
    return k(ids2d, mp)


def _k8(fidx_hbm, rows_hbm, packed_hbm, counts_hbm, fidxv, rowsg, cnt_tbl,
        tmp, pkv, sem):
    w = lax.axis_index("s")
    it = _iota16()
    _zero_vec(cnt_tbl, CTW)
    pltpu.sync_copy(fidx_hbm.at[w], fidxv)
    _fire_drain(NROW, lambda j: pltpu.make_async_copy(
        rows_hbm.at[fidxv.at[j]], rowsg.at[j], sem))

    def outer(j, _):
        def inner(k, _):
            ko = pl.ds(pl.multiple_of(k * 16, 16), 16)
            r1 = rowsg[j, ko]
            rt = jnp.where(r1 > 0, r1 - 1, MAXV)
            comp = rt * 16 + it
            sk, _sv = plsc.sort_key_val(comp, comp)
            srow = sk >> 4
            slane = sk & 15
            tmp[...] = srow
            prev = plsc.load_gather(tmp, [jnp.maximum(it - 1, 0)])
            nxt = plsc.load_gather(tmp, [jnp.minimum(it + 1, 15)])
            new_run = (it == 0) | (srow != prev)
            is_last = (it == 15) | (srow != nxt)
            segst = plsc.cummax(jnp.where(new_run, it, 0))
            occ = it - segst
            old = plsc.load_gather(cnt_tbl, [srow])
            lr = old + occ
            plsc.store_scatter(cnt_tbl, [srow], lr + 1, mask=is_last)
            pk = (srow << 14) | lr
            pos = j * 128 + k * 16 + slane
            plsc.store_scatter(pkv, [pos], pk)
            return 0

        lax.fori_loop(0, 8, inner, 0)
        return 0

    lax.fori_loop(0, NROW, outer, 0)
    pltpu.sync_copy(pkv, packed_hbm.at[pl.ds(pl.multiple_of(w * PPW, 8), PPW)])
    pltpu.sync_copy(cnt_tbl, counts_hbm.at[w])


def _run_k8(fidx, t2rows):
    k = pl.kernel(
        _k8,
        out_type=(jax.ShapeDtypeStruct((NW * PPW,), jnp.int32),
                  jax.ShapeDtypeStruct((NW, CTW), jnp.int32)),
        mesh=_mesh(),
        compiler_params=pltpu.CompilerParams(needs_layout_passes=False),
        scratch_types=[
            pltpu.VMEM((NROW, 128), jnp.int32),
            pltpu.VMEM((NROW, 128), jnp.int32),
            pltpu.VMEM((CTW,), jnp.int32),
            pltpu.VMEM((16,), jnp.int32),
            pltpu.VMEM((PPW,), jnp.int32),
            pltpu.SemaphoreType.DMA,
        ],
    )
    return k(fidx, t2rows)


def _k9(counts_hbm, prefix_hbm, nump_hbm, accv, cb):
    w = lax.axis_index("s")
    sb = w * SLAB
    _zero_vec(accv, SLAB)

    for v in range(NW):
        off = pl.multiple_of(v * CTW + sb, 8)
        pltpu.sync_copy(accv, prefix_hbm.at[pl.ds(off, SLAB)])
        pltpu.sync_copy(counts_hbm.at[v].at[pl.ds(pl.multiple_of(sb, 8), SLAB)], cb)

        def add(i, _):
            o = pl.ds(pl.multiple_of(i * 16, 16), 16)
            accv[o] = accv[o] + cb[o]
            return 0

        lax.fori_loop(0, SLAB // 16, add, 0)

    def fin(i, _):
        o = pl.ds(pl.multiple_of(i * 16, 16), 16)
        cb[o] = jnp.minimum(accv[o], MAXP)
        return 0

    lax.fori_loop(0, SLAB // 16, fin, 0)
    pltpu.sync_copy(cb, nump_hbm.at[pl.ds(pl.multiple_of(sb, 8), SLAB)])


def _run_k9(counts):
    k = pl.kernel(
        _k9,
        out_type=(jax.ShapeDtypeStruct((NW * CTW,), jnp.int32),
                  jax.ShapeDtypeStruct((CTW,), jnp.int32)),
        mesh=_mesh(),
        compiler_params=pltpu.CompilerParams(needs_layout_passes=False),
        scratch_types=[
            pltpu.VMEM((SLAB,), jnp.int32),
            pltpu.VMEM((SLAB,), jnp.int32),
        ],
    )
    return k(counts)


def _k10(px_hbm, py_hbm, pz_hbm, pw_hbm, pcrd_hbm, packed_hbm, prefix_hbm,
         vox_hbm, coors_hbm,
         zbf, zbi, pkv, cand, pidx, pgi, pg,
         vi0, vi1, vi2, vi3, ci0, ci1, ci2,
         px0, px1, px2, px3, cv0, cv1, cv2, sem):
    w = lax.axis_index("s")
    it = _iota16()
    _zero_vec(zbf, 2048)
    _zero_vec(zbi, 2048)
    for c in range(10):
        pltpu.sync_copy(zbf, vox_hbm.at[pl.ds(
            pl.multiple_of(w * 20480 + c * 2048, 8), 2048)])
    pltpu.sync_copy(zbi, coors_hbm.at[pl.ds(
        pl.multiple_of(w * 3072, 8), 2048)])
    pltpu.sync_copy(zbi.at[pl.ds(0, 1024)], coors_hbm.at[pl.ds(
        pl.multiple_of(w * 3072 + 2048, 8), 1024)])
    plsc.subcore_barrier()

    for (r0, nr) in ((0, 24), (24, 24), (48, 24), (72, 26)):
        base = w * PPW + r0 * 128
        npts = nr * 128
        pltpu.sync_copy(packed_hbm.at[pl.ds(pl.multiple_of(base, 8), npts)],
                        pkv.at[pl.ds(0, npts)])

        def pass_a(i, cnt):
            o = pl.ds(pl.multiple_of(i * 16, 16), 16)
            pk = pkv[o]
            rt = pk >> 14
            okm = rt < MAXV
            oki = okm.astype(jnp.int32)
            tgt = cnt + plsc.cumsum(oki) - 1
            pos16 = i * 16 + it
            plsc.store_scatter(cand, [jnp.where(okm, tgt, 3455)], pos16,
                               mask=okm)
            return cnt + jnp.sum(oki)

        cnt = lax.fori_loop(0, nr * 8, pass_a, jnp.int32(0))
        nch = (cnt + 127) >> 7

        def pass_b(j, _):
            for k in range(8):
                ko = pl.ds(pl.multiple_of(k * 16, 16), 16)
                slot = j * 128 + k * 16 + it
                msk = slot < cnt
                cl = plsc.load_gather(cand, [jnp.where(msk, slot, 0)])
                cpos = jnp.where(msk, cl, 0)
                pk = plsc.load_gather(pkv, [cpos])
                rt = pk >> 14
                pgi[j, ko] = jnp.where(msk, w * CTW + rt, w * CTW)
                pidx[j, ko] = jnp.where(msk, base + cpos, base)
            pltpu.make_async_copy(
                prefix_hbm.at[pgi.at[j]], pg.at[j], sem).start()
            pltpu.make_async_copy(
                prefix_hbm.at[pgi.at[j]], pg.at[j], sem).wait()
            return 0

        lax.fori_loop(0, nch, pass_b, 0)

        def gfeat(j, _):
            srcs = ((px_hbm, px0), (py_hbm, px1), (pz_hbm, px2), (pw_hbm, px3))
            for srcr, dst in srcs:
                pltpu.make_async_copy(
                    srcr.at[pidx.at[j]], dst.at[j], sem).start()
            pltpu.make_async_copy(
                pcrd_hbm.at[pidx.at[j]], cv0.at[j], sem).start()
            for srcr, dst in srcs:
                pltpu.make_async_copy(
                    srcr.at[pidx.at[j]], dst.at[j], sem).wait()
            pltpu.make_async_copy(
                pcrd_hbm.at[pidx.at[j]], cv0.at[j], sem).wait()
            return 0

        lax.fori_loop(0, nch, gfeat, 0)

        def pass_c(j, _):
            for k in range(8):
                ko = pl.ds(pl.multiple_of(k * 16, 16), 16)
                slot = j * 128 + k * 16 + it
                msk = slot < cnt
                cl = plsc.load_gather(cand, [jnp.where(msk, slot, 0)])
                cpos = jnp.where(msk, cl, 0)
                pk = plsc.load_gather(pkv, [cpos])
                rt = pk >> 14
                lr = pk & 16383
                g = pg[j, ko] + lr
                keepv = msk & (g < MAXP)
                dp = base + j * 128 + k * 16 + it
                vf = jnp.where(keepv, (rt * 5 + g) * 4, VDUMPW + dp * 4)
                vi0[j, ko] = vf
                vi1[j, ko] = vf + 1
                vi2[j, ko] = vf + 2
                vi3[j, ko] = vf + 3
                cf = jnp.where(msk, rt * 3, CDUMPW + dp * 3)
                ci0[j, ko] = cf
                ci1[j, ko] = cf + 1
                ci2[j, ko] = cf + 2
                pcv = cv0[j, ko]
                cv1[j, ko] = (pcv >> 11) & 2047
                cv2[j, ko] = pcv & 2047
                cv0[j, ko] = pcv >> 22
            return 0

        lax.fori_loop(0, nch, pass_c, 0)

        def scat(j, _):
            pairs = ((px0, vi0), (px1, vi1), (px2, vi2), (px3, vi3))
            cpairs = ((cv0, ci0), (cv1, ci1), (cv2, ci2))
            for srcb, idxb in pairs:
                pltpu.make_async_copy(
                    srcb.at[j], vox_hbm.at[idxb.at[j]], sem).start()
            for srcb, idxb in cpairs:
                pltpu.make_async_copy(
                    srcb.at[j], coors_hbm.at[idxb.at[j]], sem).start()
            for srcb, idxb in pairs:
                pltpu.make_async_copy(
                    srcb.at[j], vox_hbm.at[idxb.at[j]], sem).wait()
            for srcb, idxb in cpairs:
                pltpu.make_async_copy(
                    srcb.at[j], coors_hbm.at[idxb.at[j]], sem).wait()
            return 0

        lax.fori_loop(0, nch, scat, 0)


def _run_k10(px, py, pz, pw, pcf, packed, prefix):
    nb = 26
    k = pl.kernel(
        _k10,
        out_type=(jax.ShapeDtypeStruct((VWORDS,), jnp.float32),
                  jax.ShapeDtypeStruct((CWORDS,), jnp.int32)),
        mesh=_mesh(),
        compiler_params=pltpu.CompilerParams(needs_layout_passes=False),
        scratch_types=(
            [pltpu.VMEM((2048,), jnp.float32),
             pltpu.VMEM((2048,), jnp.int32),
             pltpu.VMEM((3328,), jnp.int32),
             pltpu.VMEM((3456,), jnp.int32)]
            + [pltpu.VMEM((nb, 128), jnp.int32) for _ in range(3)]
            + [pltpu.VMEM((nb, 128), jnp.int32) for _ in range(7)]
            + [pltpu.VMEM((nb, 128), jnp.float32) for _ in range(4)]
            + [pltpu.VMEM((nb, 128), jnp.int32) for _ in range(3)]
            + [pltpu.SemaphoreType.DMA]
        ),
    )
    return k(px, py, pz, pw, pcf, packed, prefix)


def kernel(input):
    points = input
    ids2d, pc2d = _compute_ids(points)
    p1 = _run_k2(ids2d)
    cnt = _run_k3(p1)
    mp = _run_k4(p1, cnt)
    t2, fidx = _run_k5(ids2d, mp)
    fcnt = _run_k6(t2)
    t2rows = _run_k7(t2, fcnt)
    packed, counts = _run_k8(fidx, t2rows)
    prefix, nump_buf = _run_k9(counts)
    pp = jnp.pad(points, ((0, NPAD - N), (0, 0)))
    vox_flat, coors_flat = _run_k10(
        pp[:, 0], pp[:, 1], pp[:, 2], pp[:, 3], pc2d.reshape(NPAD),
        packed, prefix)
    voxels = vox_flat[:MAXV * MAXP * 4].reshape(MAXV, MAXP, 4)
    coors = coors_flat[:MAXV * 3].reshape(MAXV, 3)
    return voxels, coors, nump_buf[:MAXV]


# speedup vs baseline: 217.7926x; 1.0358x over previous
import functools

import jax
import jax.numpy as jnp
from jax import lax
from jax.experimental import pallas as pl
from jax.experimental.pallas import tpu as pltpu
from jax.experimental.pallas import tpu_sc as plsc

GX, GY, GZ = 1408, 1600, 40
TOT = GX * GY * GZ
MAXV, MAXP = 16000, 5
N, NPAD = 200000, 200704
NW = 16
PPW = NPAD // NW
NROW = PPW // 128
P1S = 1409024
P1W = P1S // NW
NB_REAL = 1408000
NSLOTS = 16384
T2S = 1277952
T2W = T2S // NW
T2REAL = NSLOTS * 64
CTW = 16384
SLAB = CTW // NW
VWORDS, VDUMPW = 1146880, 320000
CWORDS, CDUMPW = 655360, 48384
HALF = PPW // 2
HROW = HALF // 128


def _mesh():
    return plsc.VectorSubcoreMesh(
        core_axis_name="c", subcore_axis_name="s", num_cores=1)


def _iota16():
    return lax.iota(jnp.int32, 16)


def _zero_vec(ref, nwords):
    z = jnp.zeros((16,), ref.dtype)

    def body(i, _):
        ref[pl.ds(pl.multiple_of(i * 16, 16), 16)] = z
        return 0

    lax.fori_loop(0, nwords // 16, body, 0)


def _fire_drain(n, mk, chunk=8):
    for c0 in range(0, n, chunk):
        descs = []
        for j in range(c0, min(n, c0 + chunk)):
            d = mk(j)
            d.start()
            descs.append(d)
        for d in descs:
            d.wait()


def _ids_body(x_ref, y_ref, z_ref, id_ref, pc_ref):
    x = x_ref[...]
    y = y_ref[...]
    z = z_ref[...]
    cx = jnp.floor((x - 0.0) / jnp.float32(0.05)).astype(jnp.int32)
    cy = jnp.floor((y - jnp.float32(-40.0)) / jnp.float32(0.05)).astype(jnp.int32)
    cz = jnp.floor((z - jnp.float32(-3.0)) / jnp.float32(0.1)).astype(jnp.int32)
    ok = (cx >= 0) & (cx < GX) & (cy >= 0) & (cy < GY) & (cz >= 0) & (cz < GZ)
    id_ref[...] = jnp.where(ok, (cz * GY + cy) * GX + cx, TOT)
    pc_ref[...] = (cz << 22) | (cy << 11) | cx


def _compute_ids(points):
    pad = NPAD - N
    x = jnp.pad(points[:, 0], (0, pad), constant_values=-1.0).reshape(1568, 128)
    y = jnp.pad(points[:, 1], (0, pad)).reshape(1568, 128)
    z = jnp.pad(points[:, 2], (0, pad)).reshape(1568, 128)
    ids, pc = pl.pallas_call(
        _ids_body,
        out_shape=(jax.ShapeDtypeStruct((1568, 128), jnp.int32),
                   jax.ShapeDtypeStruct((1568, 128), jnp.int32)),
    )(x, y, z)
    return ids.reshape(NW, PPW), pc.reshape(NW, PPW)


def _k2(ids_hbm, p1_hbm, zb, idsv, bkt, ones, sem):
    w = lax.axis_index("s")
    _zero_vec(zb, 2048)
    one = jnp.ones((16,), jnp.int32)
    for k in range(8):
        ones[pl.ds(k * 16, 16)] = one

    base = w * P1W

    _fire_drain(P1W // 2048, lambda c: pltpu.make_async_copy(
        zb, p1_hbm.at[pl.ds(pl.multiple_of(w * P1W + c * 2048, 8), 2048)],
        sem))
    plsc.subcore_barrier()
    pltpu.sync_copy(ids_hbm.at[w], idsv)

    def outer(j, _):
        def inner(k, _):
            o = pl.multiple_of(j * 128 + k * 16, 16)
            v = idsv[pl.ds(o, 16)]
            bkt[j, pl.ds(pl.multiple_of(k * 16, 16), 16)] = v >> 6
            return 0

        lax.fori_loop(0, 8, inner, 0)
        return 0

    lax.fori_loop(0, NROW, outer, 0)

    def mk(j):
        return pltpu.make_async_copy(ones, p1_hbm.at[bkt.at[j]], sem)

    _fire_drain(NROW, mk)


def _run_k2(ids2d):
    k = pl.kernel(
        _k2,
        out_type=jax.ShapeDtypeStruct((P1S,), jnp.int32),
        mesh=_mesh(),
        compiler_params=pltpu.CompilerParams(needs_layout_passes=False),
        scratch_types=[
            pltpu.VMEM((2048,), jnp.int32),
            pltpu.VMEM((PPW,), jnp.int32),
            pltpu.VMEM((NROW, 128), jnp.int32),
            pltpu.VMEM((128,), jnp.int32),
            pltpu.SemaphoreType.DMA,
        ],
    )
    return k(ids2d)


def _worker_prefix(cnt_hbm, cntv, w):
    it = _iota16()
    pltpu.sync_copy(cnt_hbm, cntv)
    totals = plsc.load_gather(cntv, [it, jnp.zeros((16,), jnp.int32)])
    return jnp.sum(jnp.where(it < w, totals, 0))


def _make_count_compact(total_words, per_w, real_limit, keep, plus_one):
    def body(tab_hbm, cnt_hbm, out_hbm, pbuf, cntv, stg):
        w = lax.axis_index("s")
        base = w * per_w
        it = _iota16()
        off = pl.ds(pl.multiple_of(base, 8), per_w)
        pltpu.sync_copy(tab_hbm.at[off], pbuf)

        def count(i, acc):
            v = pbuf[pl.ds(pl.multiple_of(i * 16, 16), 16)]
            g = base + i * 16 + it
            nz = (v != 0) & (g < real_limit)
            return acc + jnp.where(nz, 1, 0)

        acc = lax.fori_loop(0, per_w // 16, count,
                            jnp.zeros((16,), jnp.int32))
        stg[...] = jnp.broadcast_to(jnp.sum(acc), (16,))
        pltpu.sync_copy(stg, cnt_hbm.at[w])
        plsc.subcore_barrier()
        pfx = _worker_prefix(cnt_hbm, cntv, w)

        def compact(i, run):
            o = pl.ds(pl.multiple_of(i * 16, 16), 16)
            v = pbuf[o]
            g = base + i * 16 + it
            nz = (v != 0) & (g < real_limit)
            nzi = nz.astype(jnp.int32)
            cs = plsc.cumsum(nzi)
            r = run + cs - 1
            kept = nz & (r < keep)
            if plus_one:
                pbuf[o] = jnp.where(kept, r + 1, 0)
            else:
                pbuf[o] = jnp.where(kept, r, -1)
            return run + jnp.sum(nzi)

        lax.fori_loop(0, per_w // 16, compact, pfx)
        pltpu.sync_copy(pbuf, out_hbm.at[off])

    def run(tab):
        k = pl.kernel(
            body,
            out_type=(jax.ShapeDtypeStruct((16, 16), jnp.int32),
                      jax.ShapeDtypeStruct((total_words,), jnp.int32)),
            mesh=_mesh(),
            compiler_params=pltpu.CompilerParams(needs_layout_passes=False),
            scratch_types=[
                pltpu.VMEM((per_w,), jnp.int32),
                pltpu.VMEM((16, 16), jnp.int32),
                pltpu.VMEM((16,), jnp.int32),
            ],
        )
        return k(tab)

    return run


_run_k34 = _make_count_compact(P1S, P1W, NB_REAL, NSLOTS, False)
_run_k67 = _make_count_compact(T2S, T2W, T2REAL, MAXV, True)


def _k5(ids_hbm, map_hbm, t2_hbm, fidx_hbm, zb, idsv, bkt, mapg, fidxv,
        cand, fic, vc, sem):
    w = lax.axis_index("s")
    it = _iota16()
    _zero_vec(zb, 2048)
    base2 = w * (T2REAL // NW)

    _fire_drain(T2REAL // NW // 2048, lambda c: pltpu.make_async_copy(
        zb, t2_hbm.at[pl.ds(pl.multiple_of(base2 + c * 2048, 8), 2048)],
        sem))
    plsc.subcore_barrier()
    pltpu.sync_copy(ids_hbm.at[w], idsv)

    def outer(j, _):
        def inner(k, _):
            o = pl.multiple_of(j * 128 + k * 16, 16)
            v = idsv[pl.ds(o, 16)]
            bkt[j, pl.ds(pl.multiple_of(k * 16, 16), 16)] = v >> 6
            return 0

        lax.fori_loop(0, 8, inner, 0)
        return 0

    lax.fori_loop(0, NROW, outer, 0)
    _fire_drain(NROW, lambda j: pltpu.make_async_copy(
        map_hbm.at[bkt.at[j]], mapg.at[j], sem))

    def outer2(j, cnt):
        def inner(k, cnt):
            o = pl.multiple_of(j * 128 + k * 16, 16)
            ko = pl.ds(pl.multiple_of(k * 16, 16), 16)
            v = idsv[pl.ds(o, 16)]
            s = mapg[j, ko]
            pos = w * PPW + j * 128 + k * 16 + it
            fidxv[j, ko] = jnp.where(s >= 0, s * 64 + (v & 63), T2REAL + pos)
            okm = s >= 0
            oki = okm.astype(jnp.int32)
            tgt = cnt + plsc.cumsum(oki) - 1
            lpos = j * 128 + k * 16 + it
            plsc.store_scatter(cand, [jnp.where(okm, tgt, 12671)], lpos,
                               mask=okm)
            return cnt + jnp.sum(oki)

        return lax.fori_loop(0, 8, inner, cnt)

    cnt = lax.fori_loop(0, NROW, outer2, jnp.int32(0))
    pltpu.sync_copy(fidxv, fidx_hbm.at[w])
    nch = (cnt + 127) >> 7

    def pass_s(j, _):
        for k in range(8):
            ko = pl.ds(pl.multiple_of(k * 16, 16), 16)
            slot = j * 128 + k * 16 + it
            msk = slot < cnt
            cl = plsc.load_gather(cand, [jnp.where(msk, slot, 0)])
            cpos = jnp.where(msk, cl, 0)
            fx = plsc.load_gather(fidxv, [cpos >> 7, cpos & 127])
            vv = plsc.load_gather(idsv, [cpos])
            fic[j, ko] = jnp.where(msk, fx, T2REAL + w * PPW + slot)
            vc[j, ko] = vv + 1
        pltpu.make_async_copy(vc.at[j], t2_hbm.at[fic.at[j]], sem).start()
        pltpu.make_async_copy(vc.at[j], t2_hbm.at[fic.at[j]], sem).wait()
        return 0

    lax.fori_loop(0, nch, pass_s, 0)


def _run_k5(ids2d, mp):
    k = pl.kernel(
        _k5,
        out_type=(jax.ShapeDtypeStruct((T2S,), jnp.int32),
                  jax.ShapeDtypeStruct((NW, NROW, 128), jnp.int32)),
        mesh=_mesh(),
        compiler_params=pltpu.CompilerParams(needs_layout_passes=False),
        scratch_types=[
            pltpu.VMEM((2048,), jnp.int32),
            pltpu.VMEM((PPW,), jnp.int32),
            pltpu.VMEM((NROW, 128), jnp.int32),
            pltpu.VMEM((NROW, 128), jnp.int32),
            pltpu.VMEM((NROW, 128), jnp.int32),
            pltpu.VMEM((12672,), jnp.int32),
            pltpu.VMEM((NROW, 128), jnp.int32),
            pltpu.VMEM((NROW, 128), jnp.int32),
            pltpu.SemaphoreType.DMA,
        ],
    )
    return k(ids2d, mp)


def _k8(fidx_hbm, rows_hbm, packed_hbm, counts_hbm, fidxv, rowsg, cnt_tbl,
        tmp, pkv, sem):
    w = lax.axis_index("s")
    it = _iota16()
    _zero_vec(cnt_tbl, CTW)
    pltpu.sync_copy(fidx_hbm.at[w], fidxv)
    _fire_drain(NROW, lambda j: pltpu.make_async_copy(
        rows_hbm.at[fidxv.at[j]], rowsg.at[j], sem))

    def outer(j, _):
        def inner(k, _):
            ko = pl.ds(pl.multiple_of(k * 16, 16), 16)
            r1 = rowsg[j, ko]
            rt = jnp.where(r1 > 0, r1 - 1, MAXV)
            comp = rt * 16 + it
            sk, _sv = plsc.sort_key_val(comp, comp)
            srow = sk >> 4
            slane = sk & 15
            tmp[...] = srow
            prev = plsc.load_gather(tmp, [jnp.maximum(it - 1, 0)])
            nxt = plsc.load_gather(tmp, [jnp.minimum(it + 1, 15)])
            new_run = (it == 0) | (srow != prev)
            is_last = (it == 15) | (srow != nxt)
            segst = plsc.cummax(jnp.where(new_run, it, 0))
            occ = it - segst
            old = plsc.load_gather(cnt_tbl, [srow])
            lr = old + occ
            plsc.store_scatter(cnt_tbl, [srow], lr + 1, mask=is_last)
            pk = (srow << 14) | lr
            pos = j * 128 + k * 16 + slane
            plsc.store_scatter(pkv, [pos], pk)
            return 0

        lax.fori_loop(0, 8, inner, 0)
        return 0

    lax.fori_loop(0, NROW, outer, 0)
    pltpu.sync_copy(pkv, packed_hbm.at[pl.ds(pl.multiple_of(w * PPW, 8), PPW)])
    pltpu.sync_copy(cnt_tbl, counts_hbm.at[w])


def _run_k8(fidx, t2rows):
    k = pl.kernel(
        _k8,
        out_type=(jax.ShapeDtypeStruct((NW * PPW,), jnp.int32),
                  jax.ShapeDtypeStruct((NW, CTW), jnp.int32)),
        mesh=_mesh(),
        compiler_params=pltpu.CompilerParams(needs_layout_passes=False),
        scratch_types=[
            pltpu.VMEM((NROW, 128), jnp.int32),
            pltpu.VMEM((NROW, 128), jnp.int32),
            pltpu.VMEM((CTW,), jnp.int32),
            pltpu.VMEM((16,), jnp.int32),
            pltpu.VMEM((PPW,), jnp.int32),
            pltpu.SemaphoreType.DMA,
        ],
    )
    return k(fidx, t2rows)


def _k9(counts_hbm, prefix_hbm, nump_hbm, accv, cb):
    w = lax.axis_index("s")
    sb = w * SLAB
    _zero_vec(accv, SLAB)

    for v in range(NW):
        off = pl.multiple_of(v * CTW + sb, 8)
        pltpu.sync_copy(accv, prefix_hbm.at[pl.ds(off, SLAB)])
        pltpu.sync_copy(counts_hbm.at[v].at[pl.ds(pl.multiple_of(sb, 8), SLAB)], cb)

        def add(i, _):
            o = pl.ds(pl.multiple_of(i * 16, 16), 16)
            accv[o] = accv[o] + cb[o]
            return 0

        lax.fori_loop(0, SLAB // 16, add, 0)

    def fin(i, _):
        o = pl.ds(pl.multiple_of(i * 16, 16), 16)
        cb[o] = jnp.minimum(accv[o], MAXP)
        return 0

    lax.fori_loop(0, SLAB // 16, fin, 0)
    pltpu.sync_copy(cb, nump_hbm.at[pl.ds(pl.multiple_of(sb, 8), SLAB)])


def _run_k9(counts):
    k = pl.kernel(
        _k9,
        out_type=(jax.ShapeDtypeStruct((NW * CTW,), jnp.int32),
                  jax.ShapeDtypeStruct((CTW,), jnp.int32)),
        mesh=_mesh(),
        compiler_params=pltpu.CompilerParams(needs_layout_passes=False),
        scratch_types=[
            pltpu.VMEM((SLAB,), jnp.int32),
            pltpu.VMEM((SLAB,), jnp.int32),
        ],
    )
    return k(counts)


def _k10(px_hbm, py_hbm, pz_hbm, pw_hbm, pcrd_hbm, packed_hbm, prefix_hbm,
         vox_hbm, coors_hbm,
         zbf, zbi, pkv, cand, pidx, pgi, pg,
         vi0, vi1, vi2, vi3, ci0, ci1, ci2,
         px0, px1, px2, px3, cv0, cv1, cv2, sem):
    w = lax.axis_index("s")
    it = _iota16()
    _zero_vec(zbf, 2048)
    _zero_vec(zbi, 2048)
    _fire_drain(10, lambda c: pltpu.make_async_copy(
        zbf, vox_hbm.at[pl.ds(pl.multiple_of(w * 20480 + c * 2048, 8), 2048)],
        sem))
    pltpu.sync_copy(zbi, coors_hbm.at[pl.ds(
        pl.multiple_of(w * 3072, 8), 2048)])
    pltpu.sync_copy(zbi.at[pl.ds(0, 1024)], coors_hbm.at[pl.ds(
        pl.multiple_of(w * 3072 + 2048, 8), 1024)])
    plsc.subcore_barrier()

    for (r0, nr) in ((0, 24), (24, 24), (48, 24), (72, 26)):
        base = w * PPW + r0 * 128
        npts = nr * 128
        pltpu.sync_copy(packed_hbm.at[pl.ds(pl.multiple_of(base, 8), npts)],
                        pkv.at[pl.ds(0, npts)])

        def pass_a(i, cnt):
            o = pl.ds(pl.multiple_of(i * 16, 16), 16)
            pk = pkv[o]
            rt = pk >> 14
            okm = rt < MAXV
            oki = okm.astype(jnp.int32)
            tgt = cnt + plsc.cumsum(oki) - 1
            pos16 = i * 16 + it
            plsc.store_scatter(cand, [jnp.where(okm, tgt, 3455)], pos16,
                               mask=okm)
            return cnt + jnp.sum(oki)

        cnt = lax.fori_loop(0, nr * 8, pass_a, jnp.int32(0))
        nch = (cnt + 127) >> 7

        def pass_b(j, _):
            for k in range(8):
                ko = pl.ds(pl.multiple_of(k * 16, 16), 16)
                slot = j * 128 + k * 16 + it
                msk = slot < cnt
                cl = plsc.load_gather(cand, [jnp.where(msk, slot, 0)])
                cpos = jnp.where(msk, cl, 0)
                pk = plsc.load_gather(pkv, [cpos])
                rt = pk >> 14
                pgi[j, ko] = jnp.where(msk, w * CTW + rt, w * CTW)
                pidx[j, ko] = jnp.where(msk, base + cpos, base)
            pltpu.make_async_copy(
                prefix_hbm.at[pgi.at[j]], pg.at[j], sem).start()
            pltpu.make_async_copy(
                prefix_hbm.at[pgi.at[j]], pg.at[j], sem).wait()
            return 0

        lax.fori_loop(0, nch, pass_b, 0)

        def gfeat(j, _):
            srcs = ((px_hbm, px0), (py_hbm, px1), (pz_hbm, px2), (pw_hbm, px3))
            for srcr, dst in srcs:
                pltpu.make_async_copy(
                    srcr.at[pidx.at[j]], dst.at[j], sem).start()
            pltpu.make_async_copy(
                pcrd_hbm.at[pidx.at[j]], cv0.at[j], sem).start()
            for srcr, dst in srcs:
                pltpu.make_async_copy(
                    srcr.at[pidx.at[j]], dst.at[j], sem).wait()
            pltpu.make_async_copy(
                pcrd_hbm.at[pidx.at[j]], cv0.at[j], sem).wait()
            return 0

        lax.fori_loop(0, nch, gfeat, 0)

        def pass_c(j, _):
            for k in range(8):
                ko = pl.ds(pl.multiple_of(k * 16, 16), 16)
                slot = j * 128 + k * 16 + it
                msk = slot < cnt
                cl = plsc.load_gather(cand, [jnp.where(msk, slot, 0)])
                cpos = jnp.where(msk, cl, 0)
                pk = plsc.load_gather(pkv, [cpos])
                rt = pk >> 14
                lr = pk & 16383
                g = pg[j, ko] + lr
                keepv = msk & (g < MAXP)
                dp = base + j * 128 + k * 16 + it
                vf = jnp.where(keepv, (rt * 5 + g) * 4, VDUMPW + dp * 4)
                vi0[j, ko] = vf
                vi1[j, ko] = vf + 1
                vi2[j, ko] = vf + 2
                vi3[j, ko] = vf + 3
                cf = jnp.where(msk, rt * 3, CDUMPW + dp * 3)
                ci0[j, ko] = cf
                ci1[j, ko] = cf + 1
                ci2[j, ko] = cf + 2
                pcv = cv0[j, ko]
                cv1[j, ko] = (pcv >> 11) & 2047
                cv2[j, ko] = pcv & 2047
                cv0[j, ko] = pcv >> 22
            return 0

        lax.fori_loop(0, nch, pass_c, 0)

        def scat(j, _):
            pairs = ((px0, vi0), (px1, vi1), (px2, vi2), (px3, vi3))
            cpairs = ((cv0, ci0), (cv1, ci1), (cv2, ci2))
            for srcb, idxb in pairs:
                pltpu.make_async_copy(
                    srcb.at[j], vox_hbm.at[idxb.at[j]], sem).start()
            for srcb, idxb in cpairs:
                pltpu.make_async_copy(
                    srcb.at[j], coors_hbm.at[idxb.at[j]], sem).start()
            for srcb, idxb in pairs:
                pltpu.make_async_copy(
                    srcb.at[j], vox_hbm.at[idxb.at[j]], sem).wait()
            for srcb, idxb in cpairs:
                pltpu.make_async_copy(
                    srcb.at[j], coors_hbm.at[idxb.at[j]], sem).wait()
            return 0

        lax.fori_loop(0, nch, scat, 0)


def _run_k10(px, py, pz, pw, pcf, packed, prefix):
    nb = 26
    k = pl.kernel(
        _k10,
        out_type=(jax.ShapeDtypeStruct((VWORDS,), jnp.float32),
                  jax.ShapeDtypeStruct((CWORDS,), jnp.int32)),
        mesh=_mesh(),
        compiler_params=pltpu.CompilerParams(needs_layout_passes=False),
        scratch_types=(
            [pltpu.VMEM((2048,), jnp.float32),
             pltpu.VMEM((2048,), jnp.int32),
             pltpu.VMEM((3328,), jnp.int32),
             pltpu.VMEM((3456,), jnp.int32)]
            + [pltpu.VMEM((nb, 128), jnp.int32) for _ in range(3)]
            + [pltpu.VMEM((nb, 128), jnp.int32) for _ in range(7)]
            + [pltpu.VMEM((nb, 128), jnp.float32) for _ in range(4)]
            + [pltpu.VMEM((nb, 128), jnp.int32) for _ in range(3)]
            + [pltpu.SemaphoreType.DMA]
        ),
    )
    return k(px, py, pz, pw, pcf, packed, prefix)


def kernel(input):
    points = input
    ids2d, pc2d = _compute_ids(points)
    p1 = _run_k2(ids2d)
    _, mp = _run_k34(p1)
    t2, fidx = _run_k5(ids2d, mp)
    _, t2rows = _run_k67(t2)
    packed, counts = _run_k8(fidx, t2rows)
    prefix, nump_buf = _run_k9(counts)
    pp = jnp.pad(points, ((0, NPAD - N), (0, 0)))
    vox_flat, coors_flat = _run_k10(
        pp[:, 0], pp[:, 1], pp[:, 2], pp[:, 3], pc2d.reshape(NPAD),
        packed, prefix)
    voxels = vox_flat[:MAXV * MAXP * 4].reshape(MAXV, MAXP, 4)
    coors = coors_flat[:MAXV * 3].reshape(MAXV, 3)
    return voxels, coors, nump_buf[:MAXV]
